# Initial kernel scaffold; baseline (speedup 1.0000x reference)
import functools

import jax
import jax.numpy as jnp
from jax import lax
from jax.experimental import pallas as pl
from jax.experimental.pallas import tpu as pltpu
from jax.experimental.pallas import tpu_sc as plsc

_POOL = (2, 2)

NSC = 2
NTILE = 16
LANES = 16

WSZ = 1_835_008
C_CHUNK = 2048
K_STG = 2048


@functools.lru_cache(maxsize=None)
def _build(N: int, M: int):
    assert M % WSZ == 0
    NWIN = M // WSZ
    NT = N // NTILE
    assert NT * NTILE == N
    NCH = NT // C_CHUNK
    assert NCH * C_CHUNK == NT and NCH % 2 == 0
    SLICE = WSZ // NTILE
    ZCH = 16384
    NZ = SLICE // ZCH
    assert NZ * ZCH == SLICE

    mesh = plsc.VectorSubcoreMesh(core_axis_name="c", subcore_axis_name="s")

    @functools.partial(
        pl.kernel,
        out_type=jax.ShapeDtypeStruct((M,), jnp.float32),
        mesh=mesh,
        scratch_types=[
            pltpu.VMEM((2, C_CHUNK), jnp.int32),
            pltpu.VMEM((2, C_CHUNK), jnp.float32),
            pltpu.VMEM((K_STG,), jnp.int32),
            pltpu.VMEM((K_STG,), jnp.float32),
            pltpu.VMEM((16384,), jnp.float32),
            pltpu.VMEM_SHARED((WSZ + LANES,), jnp.float32),
            pltpu.SemaphoreType.DMA,
            pltpu.SemaphoreType.DMA,
            pltpu.SemaphoreType.DMA,
            pltpu.SemaphoreType.DMA,
        ],
    )
    def scatter_add(mask_hbm, upd_hbm, out_hbm,
                    in_idx, in_val, stg_idx, stg_val, zbuf, acc,
                    s0i, s0v, s1i, s1v):
        ZCH = 16384
        NZ = SLICE // ZCH
        c = lax.axis_index("c")
        s = lax.axis_index("s")
        tbase = s * NT

        iota16 = lax.broadcasted_iota(jnp.int32, (LANES,), 0)
        dum_idx = iota16 + WSZ
        zvec = jnp.zeros((LANES,), jnp.float32)

        def zb(r, carry):
            zbuf[pl.ds(r * LANES, LANES)] = zvec
            return carry
        lax.fori_loop(0, ZCH // LANES, zb, 0)

        def refill(r, carry):
            stg_idx[pl.ds(r * LANES, LANES)] = dum_idx
            stg_val[pl.ds(r * LANES, LANES)] = zvec
            return carry
        lax.fori_loop(0, K_STG // LANES, refill, 0)

        sems = ((s0i, s0v), (s1i, s1v))

        def issue(chunk, slot):
            semi, semv = sems[slot]
            base = tbase + chunk * C_CHUNK
            pltpu.async_copy(mask_hbm.at[pl.ds(base, C_CHUNK)],
                             in_idx.at[slot], semi)
            pltpu.async_copy(upd_hbm.at[pl.ds(base, C_CHUNK)],
                             in_val.at[slot], semv)

        def wait(slot):
            semi, semv = sems[slot]
            pltpu.make_async_copy(mask_hbm.at[pl.ds(0, C_CHUNK)],
                                  in_idx.at[slot], semi).wait()
            pltpu.make_async_copy(upd_hbm.at[pl.ds(0, C_CHUNK)],
                                  in_val.at[slot], semv).wait()

        def flush_and_reset(_cur):
            pltpu.sync_copy(stg_val, acc.at[stg_idx], add=True)
            lax.fori_loop(0, K_STG // LANES, refill, 0)
            return 0

        def win_body(i, carry):
            w = c + NSC * i

            @pl.when(w < NWIN)
            def _run():
                wbase = w * WSZ

                def zc(j, cz):
                    pltpu.sync_copy(
                        zbuf, acc.at[pl.ds(s * SLICE + j * ZCH, ZCH)])
                    return cz
                lax.fori_loop(0, NZ, zc, 0)

                issue(0, 0)
                issue(1, 1)
                plsc.subcore_barrier()

                def chunk_body(t, cur):
                    for b in range(2):
                        ch = 2 * t + b
                        wait(b)

                        def vec_body(j, cur):
                            iv = in_idx[b, pl.ds(j * LANES, LANES)]
                            vv = in_val[b, pl.ds(j * LANES, LANES)]
                            loc = iv - wbase
                            m = plsc.bitcast(loc, jnp.uint32) < jnp.uint32(WSZ)
                            cur = lax.cond(cur > K_STG - LANES,
                                           flush_and_reset, lambda x: x, cur)
                            plsc.store_compressed(
                                stg_idx.at[pl.ds(cur, LANES)], loc, mask=m)
                            plsc.store_compressed(
                                stg_val.at[pl.ds(cur, LANES)], vv, mask=m)
                            return cur + jnp.sum(m.astype(jnp.int32))

                        cur = lax.fori_loop(0, C_CHUNK // LANES, vec_body, cur)

                        @pl.when(ch + 2 < NCH)
                        def _():
                            issue(ch + 2, b)
                    return cur

                cur = lax.fori_loop(0, NCH // 2, chunk_body, 0)
                flush_and_reset(cur)
                plsc.subcore_barrier()

                def oc(j, co):
                    off = s * SLICE + j * ZCH
                    pltpu.sync_copy(acc.at[pl.ds(off, ZCH)],
                                    out_hbm.at[pl.ds(wbase + off, ZCH)])
                    return co
                lax.fori_loop(0, NZ, oc, 0)

            return carry

        lax.fori_loop(0, (NWIN + NSC - 1) // NSC, win_body, 0)

    return scatter_add


@jax.jit
def kernel(updates, mask):
    B, H, W, C = updates.shape
    out_h = H * _POOL[0]
    out_w = W * _POOL[1]
    M = B * out_h * out_w * C
    N = B * H * W * C
    fn = _build(N, M)
    out = fn(mask.reshape(-1).astype(jnp.int32), updates.reshape(-1))
    return out.reshape(-1, out_h, out_w, C)


# speedup vs baseline: 2.9863x; 2.9863x over previous
import functools

import jax
import jax.numpy as jnp
from jax import lax
from jax.experimental import pallas as pl
from jax.experimental.pallas import tpu as pltpu
from jax.experimental.pallas import tpu_sc as plsc

_POOL = (2, 2)

NSC = 2
NTILE = 16
LANES = 16

WSZ = 1_835_008
C_CHUNK = 2048
K_STG = 2048


@functools.lru_cache(maxsize=None)
def _build(N: int, M: int):
    assert M % WSZ == 0
    NWIN = M // WSZ
    NT = N // NTILE
    assert NT * NTILE == N
    NCH = NT // C_CHUNK
    assert NCH * C_CHUNK == NT and NCH % 2 == 0
    SLICE = WSZ // NTILE
    ZCH = 2048
    NZ = SLICE // ZCH
    OCH = 16384
    NO = SLICE // OCH
    assert NZ * ZCH == SLICE and NO * OCH == SLICE

    mesh = plsc.VectorSubcoreMesh(core_axis_name="c", subcore_axis_name="s")

    @functools.partial(
        pl.kernel,
        out_type=jax.ShapeDtypeStruct((M,), jnp.float32),
        mesh=mesh,
        compiler_params=pltpu.CompilerParams(needs_layout_passes=False),
        scratch_types=[
            pltpu.VMEM((2, C_CHUNK), jnp.int32),
            pltpu.VMEM((2, C_CHUNK), jnp.float32),
            pltpu.VMEM((K_STG,), jnp.int32),
            pltpu.VMEM((K_STG,), jnp.float32),
            pltpu.VMEM((2048,), jnp.float32),
            pltpu.VMEM_SHARED((WSZ + LANES,), jnp.float32),
            pltpu.SemaphoreType.DMA,
            pltpu.SemaphoreType.DMA,
            pltpu.SemaphoreType.DMA,
            pltpu.SemaphoreType.DMA,
        ],
    )
    def scatter_add(mask_hbm, upd_hbm, out_hbm,
                    in_idx, in_val, stg_idx, stg_val, zbuf, acc,
                    s0i, s0v, s1i, s1v):
        c = lax.axis_index("c")
        s = lax.axis_index("s")
        tbase = s * NT

        iota16 = lax.broadcasted_iota(jnp.int32, (LANES,), 0)
        wsz_i = jnp.full((LANES,), WSZ, jnp.int32)
        wsz_u = jnp.full((LANES,), WSZ, jnp.uint32)
        dum_idx = iota16 + wsz_i
        zvec = jnp.zeros((LANES,), jnp.float32)

        def zb(r, carry):
            zbuf[pl.ds(r * LANES, LANES)] = zvec
            return carry
        lax.fori_loop(0, 2048 // LANES, zb, 0)

        def refill(r, carry):
            stg_idx[pl.ds(r * LANES, LANES)] = dum_idx
            stg_val[pl.ds(r * LANES, LANES)] = zvec
            return carry
        lax.fori_loop(0, K_STG // LANES, refill, 0)

        sems = ((s0i, s0v), (s1i, s1v))

        def issue(chunk, slot):
            semi, semv = sems[slot]
            base = tbase + chunk * C_CHUNK
            pltpu.async_copy(mask_hbm.at[pl.ds(base, C_CHUNK)],
                             in_idx.at[slot], semi)
            pltpu.async_copy(upd_hbm.at[pl.ds(base, C_CHUNK)],
                             in_val.at[slot], semv)

        def wait(slot):
            semi, semv = sems[slot]
            pltpu.make_async_copy(mask_hbm.at[pl.ds(0, C_CHUNK)],
                                  in_idx.at[slot], semi).wait()
            pltpu.make_async_copy(upd_hbm.at[pl.ds(0, C_CHUNK)],
                                  in_val.at[slot], semv).wait()

        def flush_and_reset(_cur):
            pltpu.sync_copy(stg_val, acc.at[stg_idx], add=True)
            lax.fori_loop(0, K_STG // LANES, refill, 0)
            return 0

        def win_body(i, carry):
            w = c + NSC * i

            @pl.when(w < NWIN)
            def _run():
                wbase = w * WSZ

                def zc(j, cz):
                    pltpu.sync_copy(
                        zbuf, acc.at[pl.ds(s * SLICE + j * ZCH, ZCH)])
                    return cz
                lax.fori_loop(0, NZ, zc, 0)

                issue(0, 0)
                issue(1, 1)
                plsc.subcore_barrier()

                def chunk_body(t, cur):
                    for b in range(2):
                        ch = 2 * t + b
                        wait(b)

                        wbase_v = jnp.broadcast_to(
                            wbase, (LANES,)).astype(jnp.int32)

                        def vec_body(j, cur):
                            iv = in_idx[b, pl.ds(j * LANES, LANES)]
                            vv = in_val[b, pl.ds(j * LANES, LANES)]
                            loc = iv - wbase_v
                            m = plsc.bitcast(loc, jnp.uint32) < wsz_u
                            cur = lax.cond(cur > K_STG - LANES,
                                           flush_and_reset, lambda x: x, cur)
                            mi = m.astype(jnp.int32)
                            incl = plsc.cumsum(mi)
                            curv = jnp.broadcast_to(cur, (LANES,)).astype(
                                jnp.int32)
                            offs = (curv + incl) - mi
                            plsc.store_scatter(stg_idx, [offs], loc, mask=m)
                            plsc.store_scatter(stg_val, [offs], vv, mask=m)
                            return cur + jnp.sum(mi)

                        cur = lax.fori_loop(0, C_CHUNK // LANES, vec_body, cur)

                        @pl.when(ch + 2 < NCH)
                        def _():
                            issue(ch + 2, b)
                    return cur

                cur = lax.fori_loop(0, NCH // 2, chunk_body, 0)
                flush_and_reset(cur)
                plsc.subcore_barrier()

                def oc(j, co):
                    off = s * SLICE + j * OCH
                    pltpu.sync_copy(acc.at[pl.ds(off, OCH)],
                                    out_hbm.at[pl.ds(wbase + off, OCH)])
                    return co
                lax.fori_loop(0, NO, oc, 0)

            return carry

        lax.fori_loop(0, (NWIN + NSC - 1) // NSC, win_body, 0)

    return scatter_add


@jax.jit
def kernel(updates, mask):
    B, H, W, C = updates.shape
    out_h = H * _POOL[0]
    out_w = W * _POOL[1]
    M = B * out_h * out_w * C
    N = B * H * W * C
    fn = _build(N, M)
    out = fn(mask.reshape(-1).astype(jnp.int32), updates.reshape(-1))
    return out.reshape(-1, out_h, out_w, C)


# speedup vs baseline: 4.8431x; 1.6218x over previous
import functools

import jax
import jax.numpy as jnp
from jax import lax
from jax.experimental import pallas as pl
from jax.experimental.pallas import tpu as pltpu
from jax.experimental.pallas import tpu_sc as plsc

_POOL = (2, 2)

NSC = 2
NTILE = 16
LANES = 16

WSZ = 1_835_008
C_CHUNK = 2048
K_STG = 2048


@functools.lru_cache(maxsize=None)
def _build(N: int, M: int):
    assert M % WSZ == 0
    NWIN = M // WSZ
    NT = N // NTILE
    assert NT * NTILE == N
    NCH = NT // C_CHUNK
    assert NCH * C_CHUNK == NT and NCH % 2 == 0
    SLICE = WSZ // NTILE
    ZCH = 2048
    NZ = SLICE // ZCH
    OCH = 16384
    NO = SLICE // OCH
    assert NZ * ZCH == SLICE and NO * OCH == SLICE

    mesh = plsc.VectorSubcoreMesh(core_axis_name="c", subcore_axis_name="s")

    @functools.partial(
        pl.kernel,
        out_type=jax.ShapeDtypeStruct((M,), jnp.float32),
        mesh=mesh,
        compiler_params=pltpu.CompilerParams(needs_layout_passes=False),
        scratch_types=[
            pltpu.VMEM((2, C_CHUNK), jnp.int32),
            pltpu.VMEM((2, C_CHUNK), jnp.float32),
            pltpu.VMEM((K_STG,), jnp.int32),
            pltpu.VMEM((K_STG,), jnp.float32),
            pltpu.VMEM((2048,), jnp.float32),
            pltpu.VMEM_SHARED((WSZ + LANES,), jnp.float32),
            pltpu.SemaphoreType.DMA,
            pltpu.SemaphoreType.DMA,
            pltpu.SemaphoreType.DMA,
            pltpu.SemaphoreType.DMA,
        ],
    )
    def scatter_add(mask_hbm, upd_hbm, out_hbm,
                    in_idx, in_val, stg_idx, stg_val, zbuf, acc,
                    s0i, s0v, s1i, s1v):
        c = lax.axis_index("c")
        s = lax.axis_index("s")
        tbase = s * NT

        iota16 = lax.broadcasted_iota(jnp.int32, (LANES,), 0)
        wsz_i = jnp.full((LANES,), WSZ, jnp.int32)
        wsz_u = jnp.full((LANES,), WSZ, jnp.uint32)
        dum_idx = iota16 + wsz_i
        zvec = jnp.zeros((LANES,), jnp.float32)

        def zb(r, carry):
            zbuf[pl.ds(r * LANES, LANES)] = zvec
            return carry
        lax.fori_loop(0, 2048 // LANES, zb, 0)

        def refill(r, carry):
            stg_idx[pl.ds(r * LANES, LANES)] = dum_idx
            stg_val[pl.ds(r * LANES, LANES)] = zvec
            return carry
        lax.fori_loop(0, K_STG // LANES, refill, 0)

        sems = ((s0i, s0v), (s1i, s1v))

        def issue(chunk, slot):
            semi, semv = sems[slot]
            base = tbase + chunk * C_CHUNK
            pltpu.async_copy(mask_hbm.at[pl.ds(base, C_CHUNK)],
                             in_idx.at[slot], semi)
            pltpu.async_copy(upd_hbm.at[pl.ds(base, C_CHUNK)],
                             in_val.at[slot], semv)

        def wait(slot):
            semi, semv = sems[slot]
            pltpu.make_async_copy(mask_hbm.at[pl.ds(0, C_CHUNK)],
                                  in_idx.at[slot], semi).wait()
            pltpu.make_async_copy(upd_hbm.at[pl.ds(0, C_CHUNK)],
                                  in_val.at[slot], semv).wait()

        def flush_and_reset(_cur):
            pltpu.sync_copy(stg_val, acc.at[stg_idx], add=True)
            lax.fori_loop(0, K_STG // LANES, refill, 0)
            return 0

        def win_body(i, carry):
            w = c + NSC * i

            @pl.when(w < NWIN)
            def _run():
                wbase = w * WSZ

                def zc(j, cz):
                    pltpu.sync_copy(
                        zbuf, acc.at[pl.ds(s * SLICE + j * ZCH, ZCH)])
                    return cz
                lax.fori_loop(0, NZ, zc, 0)

                issue(0, 0)
                issue(1, 1)
                plsc.subcore_barrier()

                def chunk_body(t, cur):
                    for b in range(2):
                        ch = 2 * t + b
                        wait(b)

                        wbase_v = jnp.broadcast_to(
                            wbase, (LANES,)).astype(jnp.int32)

                        def append(cur, iv, vv):
                            loc = iv - wbase_v
                            m = plsc.bitcast(loc, jnp.uint32) < wsz_u
                            mi = m.astype(jnp.int32)
                            incl = plsc.cumsum(mi)
                            pc = plsc.all_reduce_population_count(m)
                            curv = jnp.broadcast_to(cur, (LANES,)).astype(
                                jnp.int32)
                            offs = (curv + incl) - mi
                            plsc.store_scatter(stg_idx, [offs], loc, mask=m)
                            plsc.store_scatter(stg_val, [offs], vv, mask=m)
                            return cur + pc[0]

                        def vec_body(j, cur):
                            cur = lax.cond(cur > K_STG - 2 * LANES,
                                           flush_and_reset, lambda x: x, cur)
                            iv0 = in_idx[b, pl.ds(2 * j * LANES, LANES)]
                            vv0 = in_val[b, pl.ds(2 * j * LANES, LANES)]
                            iv1 = in_idx[b, pl.ds((2 * j + 1) * LANES, LANES)]
                            vv1 = in_val[b, pl.ds((2 * j + 1) * LANES, LANES)]
                            cur = append(cur, iv0, vv0)
                            cur = append(cur, iv1, vv1)
                            return cur

                        cur = lax.fori_loop(0, C_CHUNK // (2 * LANES),
                                            vec_body, cur)

                        @pl.when(ch + 2 < NCH)
                        def _():
                            issue(ch + 2, b)
                    return cur

                cur = lax.fori_loop(0, NCH // 2, chunk_body, 0)
                flush_and_reset(cur)
                plsc.subcore_barrier()

                def oc(j, co):
                    off = s * SLICE + j * OCH
                    pltpu.sync_copy(acc.at[pl.ds(off, OCH)],
                                    out_hbm.at[pl.ds(wbase + off, OCH)])
                    return co
                lax.fori_loop(0, NO, oc, 0)

            return carry

        lax.fori_loop(0, (NWIN + NSC - 1) // NSC, win_body, 0)

    return scatter_add


@jax.jit
def kernel(updates, mask):
    B, H, W, C = updates.shape
    out_h = H * _POOL[0]
    out_w = W * _POOL[1]
    M = B * out_h * out_w * C
    N = B * H * W * C
    fn = _build(N, M)
    out = fn(mask.reshape(-1).astype(jnp.int32), updates.reshape(-1))
    return out.reshape(-1, out_h, out_w, C)


# speedup vs baseline: 8.4141x; 1.7373x over previous
import functools

import jax
import jax.numpy as jnp
from jax import lax
from jax.experimental import pallas as pl
from jax.experimental.pallas import tpu as pltpu
from jax.experimental.pallas import tpu_sc as plsc

_POOL = (2, 2)

NSC = 2
NTILE = 16
LANES = 16

WSZ = 1_835_008
C_CHUNK = 2048
K_STG = 2048


@functools.lru_cache(maxsize=None)
def _build(N: int, M: int):
    assert M % WSZ == 0
    NWIN = M // WSZ
    NT = N // NTILE
    assert NT * NTILE == N
    NCH = NT // C_CHUNK
    assert NCH * C_CHUNK == NT and NCH % 2 == 0
    SLICE = WSZ // NTILE
    ZCH = 2048
    NZ = SLICE // ZCH
    OCH = 16384
    NO = SLICE // OCH
    assert NZ * ZCH == SLICE and NO * OCH == SLICE

    mesh = plsc.VectorSubcoreMesh(core_axis_name="c", subcore_axis_name="s")

    @functools.partial(
        pl.kernel,
        out_type=jax.ShapeDtypeStruct((M,), jnp.float32),
        mesh=mesh,
        compiler_params=pltpu.CompilerParams(needs_layout_passes=False),
        scratch_types=[
            pltpu.VMEM((2, C_CHUNK), jnp.int32),
            pltpu.VMEM((2, C_CHUNK), jnp.float32),
            pltpu.VMEM((K_STG,), jnp.int32),
            pltpu.VMEM((K_STG,), jnp.float32),
            pltpu.VMEM((2048,), jnp.float32),
            pltpu.VMEM_SHARED((WSZ + LANES,), jnp.float32),
            pltpu.SemaphoreType.DMA,
            pltpu.SemaphoreType.DMA,
            pltpu.SemaphoreType.DMA,
            pltpu.SemaphoreType.DMA,
        ],
    )
    def scatter_add(mask_hbm, upd_hbm, out_hbm,
                    in_idx, in_val, stg_idx, stg_val, zbuf, acc,
                    s0i, s0v, s1i, s1v):
        c = lax.axis_index("c")
        s = lax.axis_index("s")
        tbase = s * NT

        iota16 = lax.broadcasted_iota(jnp.int32, (LANES,), 0)
        wsz_i = jnp.full((LANES,), WSZ, jnp.int32)
        wsz_u = jnp.full((LANES,), WSZ, jnp.uint32)
        dum_idx = iota16 + wsz_i
        zvec = jnp.zeros((LANES,), jnp.float32)

        def zb(r, carry):
            zbuf[pl.ds(r * LANES, LANES)] = zvec
            return carry
        lax.fori_loop(0, 2048 // LANES, zb, 0)

        def refill(r, carry):
            stg_idx[pl.ds(r * LANES, LANES)] = dum_idx
            stg_val[pl.ds(r * LANES, LANES)] = zvec
            return carry
        lax.fori_loop(0, K_STG // LANES, refill, 0)

        sems = ((s0i, s0v), (s1i, s1v))

        def issue(chunk, slot):
            semi, semv = sems[slot]
            base = tbase + chunk * C_CHUNK
            pltpu.async_copy(mask_hbm.at[pl.ds(base, C_CHUNK)],
                             in_idx.at[slot], semi)
            pltpu.async_copy(upd_hbm.at[pl.ds(base, C_CHUNK)],
                             in_val.at[slot], semv)

        def wait(slot):
            semi, semv = sems[slot]
            pltpu.make_async_copy(mask_hbm.at[pl.ds(0, C_CHUNK)],
                                  in_idx.at[slot], semi).wait()
            pltpu.make_async_copy(upd_hbm.at[pl.ds(0, C_CHUNK)],
                                  in_val.at[slot], semv).wait()

        def flush_and_reset(_cur):
            pltpu.sync_copy(stg_val, acc.at[stg_idx], add=True)
            lax.fori_loop(0, K_STG // LANES, refill, 0)
            return 0

        def win_body(i, carry):
            w = c + NSC * i

            @pl.when(w < NWIN)
            def _run():
                wbase = w * WSZ

                def zc(j, cz):
                    pltpu.sync_copy(
                        zbuf, acc.at[pl.ds(s * SLICE + j * ZCH, ZCH)])
                    return cz
                lax.fori_loop(0, NZ, zc, 0)

                issue(0, 0)
                issue(1, 1)
                plsc.subcore_barrier()

                def chunk_body(t, cur):
                    for b in range(2):
                        ch = 2 * t + b
                        wait(b)

                        wbase_v = jnp.broadcast_to(
                            wbase, (LANES,)).astype(jnp.int32)

                        UNROLL = 4

                        def vec_body(j, cur):
                            cur = lax.cond(cur > K_STG - UNROLL * LANES,
                                           flush_and_reset, lambda x: x, cur)
                            ivs, vvs, ms, mis, incls, pcs = \
                                [], [], [], [], [], []
                            for u in range(UNROLL):
                                d = pl.ds((UNROLL * j + u) * LANES, LANES)
                                ivs.append(in_idx[b, d])
                                vvs.append(in_val[b, d])
                            locs = [iv - wbase_v for iv in ivs]
                            for u in range(UNROLL):
                                m = (plsc.bitcast(locs[u], jnp.uint32)
                                     < wsz_u)
                                ms.append(m)
                                mis.append(m.astype(jnp.int32))
                            for u in range(UNROLL):
                                incls.append(plsc.cumsum(mis[u]))
                                pcs.append(
                                    plsc.all_reduce_population_count(ms[u]))
                            base = jnp.broadcast_to(cur, (LANES,)).astype(
                                jnp.int32)
                            for u in range(UNROLL):
                                offs = (base + incls[u]) - mis[u]
                                plsc.store_scatter(stg_idx, [offs], locs[u],
                                                   mask=ms[u])
                                plsc.store_scatter(stg_val, [offs], vvs[u],
                                                   mask=ms[u])
                                base = base + pcs[u]
                            return base[0]

                        cur = lax.fori_loop(0, C_CHUNK // (UNROLL * LANES),
                                            vec_body, cur)

                        @pl.when(ch + 2 < NCH)
                        def _():
                            issue(ch + 2, b)
                    return cur

                cur = lax.fori_loop(0, NCH // 2, chunk_body, 0)
                flush_and_reset(cur)
                plsc.subcore_barrier()

                def oc(j, co):
                    off = s * SLICE + j * OCH
                    pltpu.sync_copy(acc.at[pl.ds(off, OCH)],
                                    out_hbm.at[pl.ds(wbase + off, OCH)])
                    return co
                lax.fori_loop(0, NO, oc, 0)

            return carry

        lax.fori_loop(0, (NWIN + NSC - 1) // NSC, win_body, 0)

    return scatter_add


@jax.jit
def kernel(updates, mask):
    B, H, W, C = updates.shape
    out_h = H * _POOL[0]
    out_w = W * _POOL[1]
    M = B * out_h * out_w * C
    N = B * H * W * C
    fn = _build(N, M)
    out = fn(mask.reshape(-1).astype(jnp.int32), updates.reshape(-1))
    return out.reshape(-1, out_h, out_w, C)


# speedup vs baseline: 15.9344x; 1.8938x over previous
import functools

import jax
import jax.numpy as jnp
from jax import lax
from jax.experimental import pallas as pl
from jax.experimental.pallas import tpu as pltpu
from jax.experimental.pallas import tpu_sc as plsc

_POOL = (2, 2)

NSC = 2
NTILE = 16
NTG = 32
LANES = 16

WSZ = 1_835_008
NGRP = 7
WPG = 3
GRP = WSZ * WPG
KBLK = 2048
K_STG = 2048
C1 = 3072
GBUF = 5376
NBLK_CAP = 148
REG = NBLK_CAP * KBLK


def _mulshift_div(x, mul, shift):
    mv = jnp.full((LANES,), mul, jnp.int32)
    return lax.shift_right_logical(
        x * mv, jnp.full((LANES,), shift, jnp.int32))


@functools.lru_cache(maxsize=None)
def _build_phase1(N: int, M: int):
    NT32 = N // NTG
    assert NT32 * NTG == N
    NCH = NT32 // C1
    assert NCH * C1 == NT32 and NCH % 2 == 0
    SSZ = NGRP * NTG * REG

    mesh = plsc.VectorSubcoreMesh(core_axis_name="c", subcore_axis_name="s")

    @functools.partial(
        pl.kernel,
        out_type=(jax.ShapeDtypeStruct((SSZ,), jnp.int32),
                  jax.ShapeDtypeStruct((SSZ,), jnp.float32),
                  jax.ShapeDtypeStruct((NTG * LANES,), jnp.int32)),
        mesh=mesh,
        compiler_params=pltpu.CompilerParams(needs_layout_passes=False),
        scratch_types=[
            pltpu.VMEM((2, C1), jnp.int32),
            pltpu.VMEM((2, C1), jnp.float32),
            pltpu.VMEM((NGRP * GBUF,), jnp.int32),
            pltpu.VMEM((NGRP * GBUF,), jnp.float32),
            pltpu.VMEM((LANES,), jnp.int32),
            pltpu.SMEM((8,), jnp.int32),
            pltpu.SemaphoreType.DMA,
            pltpu.SemaphoreType.DMA,
            pltpu.SemaphoreType.DMA,
            pltpu.SemaphoreType.DMA,
        ],
    )
    def partition(mask_hbm, upd_hbm, sidx_hbm, sval_hbm, cnt_hbm,
                  in_idx, in_val, gb_idx, gb_val, cnt_v, blk_s,
                  s0i, s0v, s1i, s1v):
        c = lax.axis_index("c")
        s = lax.axis_index("s")
        tg = c * NTILE + s
        tbase = tg * NT32

        iota16 = lax.broadcasted_iota(jnp.int32, (LANES,), 0)
        dum_idx = iota16 + jnp.full((LANES,), GRP, jnp.int32)
        grp_v = jnp.full((LANES,), GRP, jnp.int32)

        for gg in range(NGRP):
            blk_s[gg] = 0

        sems = ((s0i, s0v), (s1i, s1v))

        def issue(chunk, slot):
            semi, semv = sems[slot]
            base = tbase + chunk * C1
            pltpu.async_copy(mask_hbm.at[pl.ds(base, C1)],
                             in_idx.at[slot], semi)
            pltpu.async_copy(upd_hbm.at[pl.ds(base, C1)],
                             in_val.at[slot], semv)

        def wait(slot):
            semi, semv = sems[slot]
            pltpu.make_async_copy(mask_hbm.at[pl.ds(0, C1)],
                                  in_idx.at[slot], semi).wait()
            pltpu.make_async_copy(upd_hbm.at[pl.ds(0, C1)],
                                  in_val.at[slot], semv).wait()

        issue(0, 0)
        issue(1, 1)

        def chunk_body(t, curs):
            for b in range(2):
                ch = 2 * t + b
                wait(b)

                def vec_body(j, curs):
                    iv = in_idx[b, pl.ds(j * LANES, LANES)]
                    vv = in_val[b, pl.ds(j * LANES, LANES)]
                    gvec = _mulshift_div(
                        lax.shift_right_logical(
                            iv, jnp.full((LANES,), 18, jnp.int32)),
                        3121, 16)
                    glocal = iv - gvec * grp_v
                    newcurs = []
                    for gg in range(NGRP):
                        m = gvec == jnp.full((LANES,), gg, jnp.int32)
                        mi = m.astype(jnp.int32)
                        incl = plsc.cumsum(mi)
                        pc = plsc.all_reduce_population_count(m)
                        offs = (curs[gg] + incl) - mi
                        plsc.store_scatter(gb_idx, [offs], glocal, mask=m)
                        plsc.store_scatter(gb_val, [offs], vv, mask=m)
                        newcurs.append(curs[gg] + pc)
                    return tuple(newcurs)

                curs = lax.fori_loop(0, C1 // LANES, vec_body, curs)

                @pl.when(ch + 2 < NCH)
                def _():
                    issue(ch + 2, b)

                newcurs = []
                for gg in range(NGRP):
                    gb0 = gg * GBUF
                    cur = curs[gg][0] - gb0
                    rbase = (gg * NTG + tg) * REG

                    def flush_blk(f):
                        @pl.when(cur >= (f + 1) * KBLK)
                        def _():
                            nb = blk_s[gg]
                            pltpu.sync_copy(
                                gb_idx.at[pl.ds(gb0 + f * KBLK, KBLK)],
                                sidx_hbm.at[pl.ds(rbase + nb * KBLK, KBLK)])
                            pltpu.sync_copy(
                                gb_val.at[pl.ds(gb0 + f * KBLK, KBLK)],
                                sval_hbm.at[pl.ds(rbase + nb * KBLK, KBLK)])
                            blk_s[gg] = nb + 1

                    flush_blk(0)
                    flush_blk(1)
                    nf = cur // KBLK
                    rem = cur - nf * KBLK

                    @pl.when(nf > 0)
                    def _():
                        def mv(r, carry):
                            d = pl.ds(gb0 + nf * KBLK + r * LANES, LANES)
                            gb_idx[pl.ds(gb0 + r * LANES, LANES)] = \
                                gb_idx[d]
                            gb_val[pl.ds(gb0 + r * LANES, LANES)] = \
                                gb_val[d]
                            return carry
                        lax.fori_loop(0, (rem + LANES - 1) // LANES, mv, 0)

                    newcurs.append(jnp.broadcast_to(
                        gb0 + rem, (LANES,)).astype(jnp.int32))
                curs = tuple(newcurs)
            return curs

        zcur = tuple(
            jnp.full((LANES,), gg * GBUF, jnp.int32) for gg in range(NGRP))
        curs = lax.fori_loop(0, NCH // 2, chunk_body, zcur)

        for gg in range(NGRP):
            gb0 = gg * GBUF
            cur = curs[gg][0] - gb0
            rbase = (gg * NTG + tg) * REG
            gb_idx[pl.ds(gb0 + cur, LANES)] = dum_idx

            def pad(r, carry):
                gb_idx[pl.ds(gb0 + r * LANES, LANES)] = dum_idx
                return carry
            lax.fori_loop(cur // LANES + 1, KBLK // LANES, pad, 0)

            @pl.when(cur > 0)
            def _():
                nb = blk_s[gg]
                pltpu.sync_copy(gb_idx.at[pl.ds(gb0, KBLK)],
                                sidx_hbm.at[pl.ds(rbase + nb * KBLK, KBLK)])
                pltpu.sync_copy(gb_val.at[pl.ds(gb0, KBLK)],
                                sval_hbm.at[pl.ds(rbase + nb * KBLK, KBLK)])
                blk_s[gg] = nb + 1

        cnt_vec = jnp.zeros((LANES,), jnp.int32)
        for gg in range(NGRP):
            sel = iota16 == jnp.full((LANES,), gg, jnp.int32)
            cnt_vec = jnp.where(
                sel,
                jnp.broadcast_to(blk_s[gg], (LANES,)).astype(jnp.int32),
                cnt_vec)
        cnt_v[pl.ds(0, LANES)] = cnt_vec
        pltpu.sync_copy(cnt_v, cnt_hbm.at[pl.ds(tg * LANES, LANES)])

    return partition


@functools.lru_cache(maxsize=None)
def _build_phase2(N: int, M: int):
    NWIN = M // WSZ
    assert NWIN == NGRP * WPG
    SLICE = WSZ // NTILE
    ZCH = 2048
    NZ = SLICE // ZCH
    OCH = 16384
    NO = SLICE // OCH
    SSZ = NGRP * NTG * REG

    mesh = plsc.VectorSubcoreMesh(core_axis_name="c", subcore_axis_name="s")

    @functools.partial(
        pl.kernel,
        out_type=jax.ShapeDtypeStruct((M,), jnp.float32),
        mesh=mesh,
        compiler_params=pltpu.CompilerParams(needs_layout_passes=False),
        scratch_types=[
            pltpu.VMEM((2, KBLK), jnp.int32),
            pltpu.VMEM((2, KBLK), jnp.float32),
            pltpu.VMEM((K_STG,), jnp.int32),
            pltpu.VMEM((K_STG,), jnp.float32),
            pltpu.VMEM((2048,), jnp.float32),
            pltpu.VMEM((NTG * LANES,), jnp.int32),
            pltpu.VMEM_SHARED((WSZ + LANES,), jnp.float32),
            pltpu.SemaphoreType.DMA,
            pltpu.SemaphoreType.DMA,
            pltpu.SemaphoreType.DMA,
            pltpu.SemaphoreType.DMA,
        ],
    )
    def accumulate(sidx_hbm, sval_hbm, cnt_hbm, out_hbm,
                   rd_idx, rd_val, stg_idx, stg_val, zbuf, cnt_v, acc,
                   s0i, s0v, s1i, s1v):
        c = lax.axis_index("c")
        s = lax.axis_index("s")

        iota16 = lax.broadcasted_iota(jnp.int32, (LANES,), 0)
        wsz_i = jnp.full((LANES,), WSZ, jnp.int32)
        wsz_u = jnp.full((LANES,), WSZ, jnp.uint32)
        dum_idx = iota16 + wsz_i
        zvec = jnp.zeros((LANES,), jnp.float32)

        def zb(r, carry):
            zbuf[pl.ds(r * LANES, LANES)] = zvec
            return carry
        lax.fori_loop(0, 2048 // LANES, zb, 0)

        def refill(r, carry):
            stg_idx[pl.ds(r * LANES, LANES)] = dum_idx
            stg_val[pl.ds(r * LANES, LANES)] = zvec
            return carry
        lax.fori_loop(0, K_STG // LANES, refill, 0)

        pltpu.sync_copy(cnt_hbm, cnt_v)

        sems = ((s0i, s0v), (s1i, s1v))

        def flush_and_reset(_cur):
            pltpu.sync_copy(stg_val, acc.at[stg_idx], add=True)
            lax.fori_loop(0, K_STG // LANES, refill, 0)
            return 0

        def win_body(i, carry):
            w = c + NSC * i

            @pl.when(w < NWIN)
            def _run():
                wbase = w * WSZ
                g = (w * 10923) >> 15
                winoff = wbase - g * GRP
                winoff_v = jnp.broadcast_to(winoff, (LANES,)).astype(
                    jnp.int32)

                r0 = 2 * s
                gv = jnp.broadcast_to(g, (LANES,)).astype(jnp.int32)
                row0 = cnt_v[pl.ds(r0 * LANES, LANES)]
                row1 = cnt_v[pl.ds((r0 + 1) * LANES, LANES)]
                gsel = iota16 == gv
                zi = jnp.zeros((LANES,), jnp.int32)
                n0 = jnp.sum(jnp.where(gsel, row0, zi))
                n1 = jnp.sum(jnp.where(gsel, row1, zi))
                tot = n0 + n1
                a0 = (g * NTG + r0) * REG
                a1 = (g * NTG + r0 + 1) * REG

                def baddr(blk):
                    return jnp.where(blk < n0,
                                     a0 + blk * KBLK,
                                     a1 + (blk - n0) * KBLK)

                def issue(blk, slot):
                    semi, semv = sems[slot]
                    ba = baddr(blk)
                    pltpu.async_copy(sidx_hbm.at[pl.ds(ba, KBLK)],
                                     rd_idx.at[slot], semi)
                    pltpu.async_copy(sval_hbm.at[pl.ds(ba, KBLK)],
                                     rd_val.at[slot], semv)

                def wait(slot):
                    semi, semv = sems[slot]
                    pltpu.make_async_copy(sidx_hbm.at[pl.ds(0, KBLK)],
                                          rd_idx.at[slot], semi).wait()
                    pltpu.make_async_copy(sval_hbm.at[pl.ds(0, KBLK)],
                                          rd_val.at[slot], semv).wait()

                def zc(j, cz):
                    pltpu.sync_copy(
                        zbuf, acc.at[pl.ds(s * SLICE + j * ZCH, ZCH)])
                    return cz
                lax.fori_loop(0, NZ, zc, 0)

                @pl.when(tot > 0)
                def _():
                    issue(0, 0)

                @pl.when(tot > 1)
                def _():
                    issue(1, 1)

                plsc.subcore_barrier()

                UNROLL = 4

                def blk_body(t, cur):
                    for b in range(2):
                        ch = 2 * t + b

                        def process(cur):
                            wait(b)

                            def vec_body(j, cur):
                                cur = lax.cond(
                                    cur > K_STG - UNROLL * LANES,
                                    flush_and_reset, lambda x: x, cur)
                                ivs, vvs, ms, mis = [], [], [], []
                                incls, pcs = [], []
                                for u in range(UNROLL):
                                    d = pl.ds((UNROLL * j + u) * LANES,
                                              LANES)
                                    ivs.append(rd_idx[b, d])
                                    vvs.append(rd_val[b, d])
                                locs = [iv - winoff_v for iv in ivs]
                                for u in range(UNROLL):
                                    m = (plsc.bitcast(locs[u], jnp.uint32)
                                         < wsz_u)
                                    ms.append(m)
                                    mis.append(m.astype(jnp.int32))
                                for u in range(UNROLL):
                                    incls.append(plsc.cumsum(mis[u]))
                                    pcs.append(
                                        plsc.all_reduce_population_count(
                                            ms[u]))
                                base = jnp.broadcast_to(
                                    cur, (LANES,)).astype(jnp.int32)
                                for u in range(UNROLL):
                                    offs = (base + incls[u]) - mis[u]
                                    plsc.store_scatter(
                                        stg_idx, [offs], locs[u],
                                        mask=ms[u])
                                    plsc.store_scatter(
                                        stg_val, [offs], vvs[u],
                                        mask=ms[u])
                                    base = base + pcs[u]
                                return base[0]

                            cur = lax.fori_loop(
                                0, KBLK // (UNROLL * LANES), vec_body, cur)

                            @pl.when(ch + 2 < tot)
                            def _():
                                issue(ch + 2, b)
                            return cur

                        cur = lax.cond(ch < tot, process,
                                       lambda x: x, cur)
                    return cur

                cur = lax.fori_loop(0, (tot + 1) // 2, blk_body, 0)
                flush_and_reset(cur)
                plsc.subcore_barrier()

                def oc(j, co):
                    off = s * SLICE + j * OCH
                    pltpu.sync_copy(acc.at[pl.ds(off, OCH)],
                                    out_hbm.at[pl.ds(wbase + off, OCH)])
                    return co
                lax.fori_loop(0, NO, oc, 0)

            return carry

        lax.fori_loop(0, (NWIN + NSC - 1) // NSC, win_body, 0)

    return accumulate


@jax.jit
def kernel(updates, mask):
    B, H, W, C = updates.shape
    out_h = H * _POOL[0]
    out_w = W * _POOL[1]
    M = B * out_h * out_w * C
    N = B * H * W * C
    p1 = _build_phase1(N, M)
    p2 = _build_phase2(N, M)
    sidx, sval, cnts = p1(mask.reshape(-1).astype(jnp.int32),
                          updates.reshape(-1))
    out = p2(sidx, sval, cnts)
    return out.reshape(-1, out_h, out_w, C)


# speedup vs baseline: 17.0086x; 1.0674x over previous
import functools

import jax
import jax.numpy as jnp
from jax import lax
from jax.experimental import pallas as pl
from jax.experimental.pallas import tpu as pltpu
from jax.experimental.pallas import tpu_sc as plsc

_POOL = (2, 2)

NSC = 2
NTILE = 16
NTG = 32
LANES = 16

WSZ = 1_835_008
NGRP = 7
WPG = 3
GRP = WSZ * WPG
KBLK = 2048
K_STG = 2048
C1 = 3072
GBUF = 5376
NBLK_CAP = 148
REG = NBLK_CAP * KBLK


def _mulshift_div(x, mul, shift):
    mv = jnp.full((LANES,), mul, jnp.int32)
    return lax.shift_right_logical(
        x * mv, jnp.full((LANES,), shift, jnp.int32))


@functools.lru_cache(maxsize=None)
def _build_phase1(N: int, M: int):
    NT32 = N // NTG
    assert NT32 * NTG == N
    NCH = NT32 // C1
    assert NCH * C1 == NT32 and NCH % 2 == 0
    SSZ = NGRP * NTG * REG

    mesh = plsc.VectorSubcoreMesh(core_axis_name="c", subcore_axis_name="s")

    @functools.partial(
        pl.kernel,
        out_type=(jax.ShapeDtypeStruct((SSZ,), jnp.int32),
                  jax.ShapeDtypeStruct((SSZ,), jnp.float32),
                  jax.ShapeDtypeStruct((NTG * LANES,), jnp.int32)),
        mesh=mesh,
        compiler_params=pltpu.CompilerParams(needs_layout_passes=False),
        scratch_types=[
            pltpu.VMEM((2, C1), jnp.int32),
            pltpu.VMEM((2, C1), jnp.float32),
            pltpu.VMEM((NGRP * GBUF,), jnp.int32),
            pltpu.VMEM((NGRP * GBUF,), jnp.float32),
            pltpu.VMEM((LANES,), jnp.int32),
            pltpu.SMEM((8,), jnp.int32),
            pltpu.SemaphoreType.DMA,
            pltpu.SemaphoreType.DMA,
            pltpu.SemaphoreType.DMA,
            pltpu.SemaphoreType.DMA,
        ],
    )
    def partition(mask_hbm, upd_hbm, sidx_hbm, sval_hbm, cnt_hbm,
                  in_idx, in_val, gb_idx, gb_val, cnt_v, blk_s,
                  s0i, s0v, s1i, s1v):
        c = lax.axis_index("c")
        s = lax.axis_index("s")
        tg = c * NTILE + s
        tbase = tg * NT32

        iota16 = lax.broadcasted_iota(jnp.int32, (LANES,), 0)
        dum_idx = iota16 + jnp.full((LANES,), GRP, jnp.int32)
        grp_v = jnp.full((LANES,), GRP, jnp.int32)

        for gg in range(NGRP):
            blk_s[gg] = 0

        sems = ((s0i, s0v), (s1i, s1v))

        def issue(chunk, slot):
            semi, semv = sems[slot]
            base = tbase + chunk * C1
            pltpu.async_copy(mask_hbm.at[pl.ds(base, C1)],
                             in_idx.at[slot], semi)
            pltpu.async_copy(upd_hbm.at[pl.ds(base, C1)],
                             in_val.at[slot], semv)

        def wait(slot):
            semi, semv = sems[slot]
            pltpu.make_async_copy(mask_hbm.at[pl.ds(0, C1)],
                                  in_idx.at[slot], semi).wait()
            pltpu.make_async_copy(upd_hbm.at[pl.ds(0, C1)],
                                  in_val.at[slot], semv).wait()

        issue(0, 0)
        issue(1, 1)

        def chunk_body(t, curs):
            for b in range(2):
                ch = 2 * t + b
                wait(b)

                def vec_body(j, curs):
                    U1 = 2
                    ivs, vvs, gvecs, glocals = [], [], [], []
                    for u in range(U1):
                        d = pl.ds((U1 * j + u) * LANES, LANES)
                        ivs.append(in_idx[b, d])
                        vvs.append(in_val[b, d])
                    for u in range(U1):
                        gv = _mulshift_div(
                            lax.shift_right_logical(
                                ivs[u], jnp.full((LANES,), 18, jnp.int32)),
                            3121, 16)
                        gvecs.append(gv)
                        glocals.append(ivs[u] - gv * grp_v)
                    newcurs = []
                    for gg in range(NGRP):
                        ggv = jnp.full((LANES,), gg, jnp.int32)
                        ms = [gvecs[u] == ggv for u in range(U1)]
                        mis = [m.astype(jnp.int32) for m in ms]
                        incls = [plsc.cumsum(mi) for mi in mis]
                        pcs = [plsc.all_reduce_population_count(m)
                               for m in ms]
                        base = curs[gg]
                        for u in range(U1):
                            offs = (base + incls[u]) - mis[u]
                            plsc.store_scatter(gb_idx, [offs], glocals[u],
                                               mask=ms[u])
                            plsc.store_scatter(gb_val, [offs], vvs[u],
                                               mask=ms[u])
                            base = base + pcs[u]
                        newcurs.append(base)
                    return tuple(newcurs)

                curs = lax.fori_loop(0, C1 // (2 * LANES), vec_body, curs)

                @pl.when(ch + 2 < NCH)
                def _():
                    issue(ch + 2, b)

                newcurs = []
                for gg in range(NGRP):
                    gb0 = gg * GBUF
                    cur = curs[gg][0] - gb0
                    rbase = (gg * NTG + tg) * REG

                    def flush_blk(f):
                        @pl.when(cur >= (f + 1) * KBLK)
                        def _():
                            nb = blk_s[gg]
                            pltpu.sync_copy(
                                gb_idx.at[pl.ds(gb0 + f * KBLK, KBLK)],
                                sidx_hbm.at[pl.ds(rbase + nb * KBLK, KBLK)])
                            pltpu.sync_copy(
                                gb_val.at[pl.ds(gb0 + f * KBLK, KBLK)],
                                sval_hbm.at[pl.ds(rbase + nb * KBLK, KBLK)])
                            blk_s[gg] = nb + 1

                    flush_blk(0)
                    flush_blk(1)
                    nf = cur // KBLK
                    rem = cur - nf * KBLK

                    @pl.when(nf > 0)
                    def _():
                        def mv(r, carry):
                            d = pl.ds(gb0 + nf * KBLK + r * LANES, LANES)
                            gb_idx[pl.ds(gb0 + r * LANES, LANES)] = \
                                gb_idx[d]
                            gb_val[pl.ds(gb0 + r * LANES, LANES)] = \
                                gb_val[d]
                            return carry
                        lax.fori_loop(0, (rem + LANES - 1) // LANES, mv, 0)

                    newcurs.append(jnp.broadcast_to(
                        gb0 + rem, (LANES,)).astype(jnp.int32))
                curs = tuple(newcurs)
            return curs

        zcur = tuple(
            jnp.full((LANES,), gg * GBUF, jnp.int32) for gg in range(NGRP))
        curs = lax.fori_loop(0, NCH // 2, chunk_body, zcur)

        for gg in range(NGRP):
            gb0 = gg * GBUF
            cur = curs[gg][0] - gb0
            rbase = (gg * NTG + tg) * REG
            gb_idx[pl.ds(gb0 + cur, LANES)] = dum_idx

            def pad(r, carry):
                gb_idx[pl.ds(gb0 + r * LANES, LANES)] = dum_idx
                return carry
            lax.fori_loop(cur // LANES + 1, KBLK // LANES, pad, 0)

            @pl.when(cur > 0)
            def _():
                nb = blk_s[gg]
                pltpu.sync_copy(gb_idx.at[pl.ds(gb0, KBLK)],
                                sidx_hbm.at[pl.ds(rbase + nb * KBLK, KBLK)])
                pltpu.sync_copy(gb_val.at[pl.ds(gb0, KBLK)],
                                sval_hbm.at[pl.ds(rbase + nb * KBLK, KBLK)])
                blk_s[gg] = nb + 1

        cnt_vec = jnp.zeros((LANES,), jnp.int32)
        for gg in range(NGRP):
            sel = iota16 == jnp.full((LANES,), gg, jnp.int32)
            cnt_vec = jnp.where(
                sel,
                jnp.broadcast_to(blk_s[gg], (LANES,)).astype(jnp.int32),
                cnt_vec)
        cnt_v[pl.ds(0, LANES)] = cnt_vec
        pltpu.sync_copy(cnt_v, cnt_hbm.at[pl.ds(tg * LANES, LANES)])

    return partition


@functools.lru_cache(maxsize=None)
def _build_phase2(N: int, M: int):
    NWIN = M // WSZ
    assert NWIN == NGRP * WPG
    SLICE = WSZ // NTILE
    ZCH = 2048
    NZ = SLICE // ZCH
    OCH = 16384
    NO = SLICE // OCH
    SSZ = NGRP * NTG * REG

    mesh = plsc.VectorSubcoreMesh(core_axis_name="c", subcore_axis_name="s")

    @functools.partial(
        pl.kernel,
        out_type=jax.ShapeDtypeStruct((M,), jnp.float32),
        mesh=mesh,
        compiler_params=pltpu.CompilerParams(needs_layout_passes=False),
        scratch_types=[
            pltpu.VMEM((2, KBLK), jnp.int32),
            pltpu.VMEM((2, KBLK), jnp.float32),
            pltpu.VMEM((K_STG,), jnp.int32),
            pltpu.VMEM((K_STG,), jnp.float32),
            pltpu.VMEM((2048,), jnp.float32),
            pltpu.VMEM((NTG * LANES,), jnp.int32),
            pltpu.VMEM_SHARED((WSZ + LANES,), jnp.float32),
            pltpu.SemaphoreType.DMA,
            pltpu.SemaphoreType.DMA,
            pltpu.SemaphoreType.DMA,
            pltpu.SemaphoreType.DMA,
        ],
    )
    def accumulate(sidx_hbm, sval_hbm, cnt_hbm, out_hbm,
                   rd_idx, rd_val, stg_idx, stg_val, zbuf, cnt_v, acc,
                   s0i, s0v, s1i, s1v):
        c = lax.axis_index("c")
        s = lax.axis_index("s")

        iota16 = lax.broadcasted_iota(jnp.int32, (LANES,), 0)
        wsz_i = jnp.full((LANES,), WSZ, jnp.int32)
        wsz_u = jnp.full((LANES,), WSZ, jnp.uint32)
        dum_idx = iota16 + wsz_i
        zvec = jnp.zeros((LANES,), jnp.float32)

        def zb(r, carry):
            zbuf[pl.ds(r * LANES, LANES)] = zvec
            return carry
        lax.fori_loop(0, 2048 // LANES, zb, 0)

        def refill(r, carry):
            stg_idx[pl.ds(r * LANES, LANES)] = dum_idx
            stg_val[pl.ds(r * LANES, LANES)] = zvec
            return carry
        lax.fori_loop(0, K_STG // LANES, refill, 0)

        pltpu.sync_copy(cnt_hbm, cnt_v)

        sems = ((s0i, s0v), (s1i, s1v))

        def flush_and_reset(_cur):
            pltpu.sync_copy(stg_val, acc.at[stg_idx], add=True)
            lax.fori_loop(0, K_STG // LANES, refill, 0)
            return 0

        def win_body(i, carry):
            w = c + NSC * i

            @pl.when(w < NWIN)
            def _run():
                wbase = w * WSZ
                g = (w * 10923) >> 15
                winoff = wbase - g * GRP
                winoff_v = jnp.broadcast_to(winoff, (LANES,)).astype(
                    jnp.int32)

                r0 = 2 * s
                gv = jnp.broadcast_to(g, (LANES,)).astype(jnp.int32)
                row0 = cnt_v[pl.ds(r0 * LANES, LANES)]
                row1 = cnt_v[pl.ds((r0 + 1) * LANES, LANES)]
                gsel = iota16 == gv
                zi = jnp.zeros((LANES,), jnp.int32)
                n0 = jnp.sum(jnp.where(gsel, row0, zi))
                n1 = jnp.sum(jnp.where(gsel, row1, zi))
                tot = n0 + n1
                a0 = (g * NTG + r0) * REG
                a1 = (g * NTG + r0 + 1) * REG

                def baddr(blk):
                    return jnp.where(blk < n0,
                                     a0 + blk * KBLK,
                                     a1 + (blk - n0) * KBLK)

                def issue(blk, slot):
                    semi, semv = sems[slot]
                    ba = baddr(blk)
                    pltpu.async_copy(sidx_hbm.at[pl.ds(ba, KBLK)],
                                     rd_idx.at[slot], semi)
                    pltpu.async_copy(sval_hbm.at[pl.ds(ba, KBLK)],
                                     rd_val.at[slot], semv)

                def wait(slot):
                    semi, semv = sems[slot]
                    pltpu.make_async_copy(sidx_hbm.at[pl.ds(0, KBLK)],
                                          rd_idx.at[slot], semi).wait()
                    pltpu.make_async_copy(sval_hbm.at[pl.ds(0, KBLK)],
                                          rd_val.at[slot], semv).wait()

                def zc(j, cz):
                    pltpu.sync_copy(
                        zbuf, acc.at[pl.ds(s * SLICE + j * ZCH, ZCH)])
                    return cz
                lax.fori_loop(0, NZ, zc, 0)

                @pl.when(tot > 0)
                def _():
                    issue(0, 0)

                @pl.when(tot > 1)
                def _():
                    issue(1, 1)

                plsc.subcore_barrier()

                UNROLL = 4

                def blk_body(t, cur):
                    for b in range(2):
                        ch = 2 * t + b

                        def process(cur):
                            wait(b)

                            def vec_body(j, cur):
                                cur = lax.cond(
                                    cur > K_STG - UNROLL * LANES,
                                    flush_and_reset, lambda x: x, cur)
                                ivs, vvs, ms, mis = [], [], [], []
                                incls, pcs = [], []
                                for u in range(UNROLL):
                                    d = pl.ds((UNROLL * j + u) * LANES,
                                              LANES)
                                    ivs.append(rd_idx[b, d])
                                    vvs.append(rd_val[b, d])
                                locs = [iv - winoff_v for iv in ivs]
                                for u in range(UNROLL):
                                    m = (plsc.bitcast(locs[u], jnp.uint32)
                                         < wsz_u)
                                    ms.append(m)
                                    mis.append(m.astype(jnp.int32))
                                for u in range(UNROLL):
                                    incls.append(plsc.cumsum(mis[u]))
                                    pcs.append(
                                        plsc.all_reduce_population_count(
                                            ms[u]))
                                base = jnp.broadcast_to(
                                    cur, (LANES,)).astype(jnp.int32)
                                for u in range(UNROLL):
                                    offs = (base + incls[u]) - mis[u]
                                    plsc.store_scatter(
                                        stg_idx, [offs], locs[u],
                                        mask=ms[u])
                                    plsc.store_scatter(
                                        stg_val, [offs], vvs[u],
                                        mask=ms[u])
                                    base = base + pcs[u]
                                return base[0]

                            cur = lax.fori_loop(
                                0, KBLK // (UNROLL * LANES), vec_body, cur)

                            @pl.when(ch + 2 < tot)
                            def _():
                                issue(ch + 2, b)
                            return cur

                        cur = lax.cond(ch < tot, process,
                                       lambda x: x, cur)
                    return cur

                cur = lax.fori_loop(0, (tot + 1) // 2, blk_body, 0)
                flush_and_reset(cur)
                plsc.subcore_barrier()

                def oc(j, co):
                    off = s * SLICE + j * OCH
                    pltpu.sync_copy(acc.at[pl.ds(off, OCH)],
                                    out_hbm.at[pl.ds(wbase + off, OCH)])
                    return co
                lax.fori_loop(0, NO, oc, 0)

            return carry

        lax.fori_loop(0, (NWIN + NSC - 1) // NSC, win_body, 0)

    return accumulate


@jax.jit
def kernel(updates, mask):
    B, H, W, C = updates.shape
    out_h = H * _POOL[0]
    out_w = W * _POOL[1]
    M = B * out_h * out_w * C
    N = B * H * W * C
    p1 = _build_phase1(N, M)
    p2 = _build_phase2(N, M)
    sidx, sval, cnts = p1(mask.reshape(-1).astype(jnp.int32),
                          updates.reshape(-1))
    out = p2(sidx, sval, cnts)
    return out.reshape(-1, out_h, out_w, C)


# speedup vs baseline: 18.5395x; 1.0900x over previous
import functools

import jax
import jax.numpy as jnp
from jax import lax
from jax.experimental import pallas as pl
from jax.experimental.pallas import tpu as pltpu
from jax.experimental.pallas import tpu_sc as plsc

_POOL = (2, 2)

NSC = 2
NTILE = 16
NTG = 32
LANES = 16

WSZ = 1_835_008
NGRP = 7
WPG = 3
GRP = WSZ * WPG
KBLK = 2048
K_STG = 2048
C1 = 3072
GBUF = 5376
NBLK_CAP = 148
REG = NBLK_CAP * KBLK


def _mulshift_div(x, mul, shift):
    mv = jnp.full((LANES,), mul, jnp.int32)
    return lax.shift_right_logical(
        x * mv, jnp.full((LANES,), shift, jnp.int32))


@functools.lru_cache(maxsize=None)
def _build_phase1(N: int, M: int):
    NT32 = N // NTG
    assert NT32 * NTG == N
    NCH = NT32 // C1
    assert NCH * C1 == NT32 and NCH % 2 == 0
    SSZ = NGRP * NTG * REG

    mesh = plsc.VectorSubcoreMesh(core_axis_name="c", subcore_axis_name="s")

    @functools.partial(
        pl.kernel,
        out_type=(jax.ShapeDtypeStruct((SSZ,), jnp.int32),
                  jax.ShapeDtypeStruct((SSZ,), jnp.float32),
                  jax.ShapeDtypeStruct((NTG * LANES,), jnp.int32)),
        mesh=mesh,
        compiler_params=pltpu.CompilerParams(needs_layout_passes=False),
        scratch_types=[
            pltpu.VMEM((2, C1), jnp.int32),
            pltpu.VMEM((2, C1), jnp.float32),
            pltpu.VMEM((NGRP * GBUF,), jnp.int32),
            pltpu.VMEM((NGRP * GBUF,), jnp.float32),
            pltpu.VMEM((LANES,), jnp.int32),
            pltpu.SMEM((8,), jnp.int32),
            pltpu.SemaphoreType.DMA,
            pltpu.SemaphoreType.DMA,
            pltpu.SemaphoreType.DMA,
            pltpu.SemaphoreType.DMA,
        ],
    )
    def partition(mask_hbm, upd_hbm, sidx_hbm, sval_hbm, cnt_hbm,
                  in_idx, in_val, gb_idx, gb_val, cnt_v, blk_s,
                  s0i, s0v, s1i, s1v):
        c = lax.axis_index("c")
        s = lax.axis_index("s")
        tg = c * NTILE + s
        tbase = tg * NT32

        iota16 = lax.broadcasted_iota(jnp.int32, (LANES,), 0)
        dum_idx = iota16 + jnp.full((LANES,), GRP, jnp.int32)
        grp_v = jnp.full((LANES,), GRP, jnp.int32)

        for gg in range(NGRP):
            blk_s[gg] = 0

        sems = ((s0i, s0v), (s1i, s1v))

        def issue(chunk, slot):
            semi, semv = sems[slot]
            base = tbase + chunk * C1
            pltpu.async_copy(mask_hbm.at[pl.ds(base, C1)],
                             in_idx.at[slot], semi)
            pltpu.async_copy(upd_hbm.at[pl.ds(base, C1)],
                             in_val.at[slot], semv)

        def wait(slot):
            semi, semv = sems[slot]
            pltpu.make_async_copy(mask_hbm.at[pl.ds(0, C1)],
                                  in_idx.at[slot], semi).wait()
            pltpu.make_async_copy(upd_hbm.at[pl.ds(0, C1)],
                                  in_val.at[slot], semv).wait()

        issue(0, 0)
        issue(1, 1)

        def chunk_body(t, curs):
            for b in range(2):
                ch = 2 * t + b
                wait(b)

                def vec_body(j, curs):
                    U1 = 3
                    ivs, vvs, gvecs, glocals = [], [], [], []
                    for u in range(U1):
                        d = pl.ds((U1 * j + u) * LANES, LANES)
                        ivs.append(in_idx[b, d])
                        vvs.append(in_val[b, d])
                    for u in range(U1):
                        gv = _mulshift_div(
                            lax.shift_right_logical(
                                ivs[u], jnp.full((LANES,), 18, jnp.int32)),
                            3121, 16)
                        gvecs.append(gv)
                        glocals.append(ivs[u] - gv * grp_v)
                    newcurs = []
                    for gg in range(NGRP):
                        ggv = jnp.full((LANES,), gg, jnp.int32)
                        ms = [gvecs[u] == ggv for u in range(U1)]
                        mis = [m.astype(jnp.int32) for m in ms]
                        incls = [plsc.cumsum(mi) for mi in mis]
                        pcs = [plsc.all_reduce_population_count(m)
                               for m in ms]
                        base = curs[gg]
                        for u in range(U1):
                            offs = (base + incls[u]) - mis[u]
                            plsc.store_scatter(gb_idx, [offs], glocals[u],
                                               mask=ms[u])
                            plsc.store_scatter(gb_val, [offs], vvs[u],
                                               mask=ms[u])
                            base = base + pcs[u]
                        newcurs.append(base)
                    return tuple(newcurs)

                curs = lax.fori_loop(0, C1 // (3 * LANES), vec_body, curs)

                @pl.when(ch + 2 < NCH)
                def _():
                    issue(ch + 2, b)

                newcurs = []
                for gg in range(NGRP):
                    gb0 = gg * GBUF
                    cur = curs[gg][0] - gb0
                    rbase = (gg * NTG + tg) * REG

                    def flush_blk(f):
                        @pl.when(cur >= (f + 1) * KBLK)
                        def _():
                            nb = blk_s[gg]
                            pltpu.sync_copy(
                                gb_idx.at[pl.ds(gb0 + f * KBLK, KBLK)],
                                sidx_hbm.at[pl.ds(rbase + nb * KBLK, KBLK)])
                            pltpu.sync_copy(
                                gb_val.at[pl.ds(gb0 + f * KBLK, KBLK)],
                                sval_hbm.at[pl.ds(rbase + nb * KBLK, KBLK)])
                            blk_s[gg] = nb + 1

                    flush_blk(0)
                    flush_blk(1)
                    nf = cur // KBLK
                    rem = cur - nf * KBLK

                    @pl.when(nf > 0)
                    def _():
                        def mv(r, carry):
                            d = pl.ds(gb0 + nf * KBLK + r * LANES, LANES)
                            gb_idx[pl.ds(gb0 + r * LANES, LANES)] = \
                                gb_idx[d]
                            gb_val[pl.ds(gb0 + r * LANES, LANES)] = \
                                gb_val[d]
                            return carry
                        lax.fori_loop(0, (rem + LANES - 1) // LANES, mv, 0)

                    newcurs.append(jnp.broadcast_to(
                        gb0 + rem, (LANES,)).astype(jnp.int32))
                curs = tuple(newcurs)
            return curs

        zcur = tuple(
            jnp.full((LANES,), gg * GBUF, jnp.int32) for gg in range(NGRP))
        curs = lax.fori_loop(0, NCH // 2, chunk_body, zcur)

        for gg in range(NGRP):
            gb0 = gg * GBUF
            cur = curs[gg][0] - gb0
            rbase = (gg * NTG + tg) * REG
            gb_idx[pl.ds(gb0 + cur, LANES)] = dum_idx

            def pad(r, carry):
                gb_idx[pl.ds(gb0 + r * LANES, LANES)] = dum_idx
                return carry
            lax.fori_loop(cur // LANES + 1, KBLK // LANES, pad, 0)

            @pl.when(cur > 0)
            def _():
                nb = blk_s[gg]
                pltpu.sync_copy(gb_idx.at[pl.ds(gb0, KBLK)],
                                sidx_hbm.at[pl.ds(rbase + nb * KBLK, KBLK)])
                pltpu.sync_copy(gb_val.at[pl.ds(gb0, KBLK)],
                                sval_hbm.at[pl.ds(rbase + nb * KBLK, KBLK)])
                blk_s[gg] = nb + 1

        cnt_vec = jnp.zeros((LANES,), jnp.int32)
        for gg in range(NGRP):
            sel = iota16 == jnp.full((LANES,), gg, jnp.int32)
            cnt_vec = jnp.where(
                sel,
                jnp.broadcast_to(blk_s[gg], (LANES,)).astype(jnp.int32),
                cnt_vec)
        cnt_v[pl.ds(0, LANES)] = cnt_vec
        pltpu.sync_copy(cnt_v, cnt_hbm.at[pl.ds(tg * LANES, LANES)])

    return partition


@functools.lru_cache(maxsize=None)
def _build_phase2(N: int, M: int):
    NWIN = M // WSZ
    assert NWIN == NGRP * WPG
    SLICE = WSZ // NTILE
    ZCH = 2048
    NZ = SLICE // ZCH
    OCH = 16384
    NO = SLICE // OCH
    SSZ = NGRP * NTG * REG

    mesh = plsc.VectorSubcoreMesh(core_axis_name="c", subcore_axis_name="s")

    @functools.partial(
        pl.kernel,
        out_type=jax.ShapeDtypeStruct((M,), jnp.float32),
        mesh=mesh,
        compiler_params=pltpu.CompilerParams(needs_layout_passes=False),
        scratch_types=[
            pltpu.VMEM((2, KBLK), jnp.int32),
            pltpu.VMEM((2, KBLK), jnp.float32),
            pltpu.VMEM((K_STG,), jnp.int32),
            pltpu.VMEM((K_STG,), jnp.float32),
            pltpu.VMEM((2048,), jnp.float32),
            pltpu.VMEM((NTG * LANES,), jnp.int32),
            pltpu.VMEM_SHARED((WSZ + LANES,), jnp.float32),
            pltpu.SemaphoreType.DMA,
            pltpu.SemaphoreType.DMA,
            pltpu.SemaphoreType.DMA,
            pltpu.SemaphoreType.DMA,
        ],
    )
    def accumulate(sidx_hbm, sval_hbm, cnt_hbm, out_hbm,
                   rd_idx, rd_val, stg_idx, stg_val, zbuf, cnt_v, acc,
                   s0i, s0v, s1i, s1v):
        c = lax.axis_index("c")
        s = lax.axis_index("s")

        iota16 = lax.broadcasted_iota(jnp.int32, (LANES,), 0)
        wsz_i = jnp.full((LANES,), WSZ, jnp.int32)
        wsz_u = jnp.full((LANES,), WSZ, jnp.uint32)
        dum_idx = iota16 + wsz_i
        zvec = jnp.zeros((LANES,), jnp.float32)

        def zb(r, carry):
            zbuf[pl.ds(r * LANES, LANES)] = zvec
            return carry
        lax.fori_loop(0, 2048 // LANES, zb, 0)

        def refill(r, carry):
            stg_idx[pl.ds(r * LANES, LANES)] = dum_idx
            stg_val[pl.ds(r * LANES, LANES)] = zvec
            return carry
        lax.fori_loop(0, K_STG // LANES, refill, 0)

        pltpu.sync_copy(cnt_hbm, cnt_v)

        sems = ((s0i, s0v), (s1i, s1v))

        def flush_and_reset(_cur):
            pltpu.sync_copy(stg_val, acc.at[stg_idx], add=True)
            lax.fori_loop(0, K_STG // LANES, refill, 0)
            return 0

        def win_body(i, carry):
            w = c + NSC * i

            @pl.when(w < NWIN)
            def _run():
                wbase = w * WSZ
                g = (w * 10923) >> 15
                winoff = wbase - g * GRP
                winoff_v = jnp.broadcast_to(winoff, (LANES,)).astype(
                    jnp.int32)

                r0 = 2 * s
                gv = jnp.broadcast_to(g, (LANES,)).astype(jnp.int32)
                row0 = cnt_v[pl.ds(r0 * LANES, LANES)]
                row1 = cnt_v[pl.ds((r0 + 1) * LANES, LANES)]
                gsel = iota16 == gv
                zi = jnp.zeros((LANES,), jnp.int32)
                n0 = jnp.sum(jnp.where(gsel, row0, zi))
                n1 = jnp.sum(jnp.where(gsel, row1, zi))
                tot = n0 + n1
                a0 = (g * NTG + r0) * REG
                a1 = (g * NTG + r0 + 1) * REG

                def baddr(blk):
                    return jnp.where(blk < n0,
                                     a0 + blk * KBLK,
                                     a1 + (blk - n0) * KBLK)

                def issue(blk, slot):
                    semi, semv = sems[slot]
                    ba = baddr(blk)
                    pltpu.async_copy(sidx_hbm.at[pl.ds(ba, KBLK)],
                                     rd_idx.at[slot], semi)
                    pltpu.async_copy(sval_hbm.at[pl.ds(ba, KBLK)],
                                     rd_val.at[slot], semv)

                def wait(slot):
                    semi, semv = sems[slot]
                    pltpu.make_async_copy(sidx_hbm.at[pl.ds(0, KBLK)],
                                          rd_idx.at[slot], semi).wait()
                    pltpu.make_async_copy(sval_hbm.at[pl.ds(0, KBLK)],
                                          rd_val.at[slot], semv).wait()

                def zc(j, cz):
                    pltpu.sync_copy(
                        zbuf, acc.at[pl.ds(s * SLICE + j * ZCH, ZCH)])
                    return cz
                lax.fori_loop(0, NZ, zc, 0)

                @pl.when(tot > 0)
                def _():
                    issue(0, 0)

                @pl.when(tot > 1)
                def _():
                    issue(1, 1)

                plsc.subcore_barrier()

                UNROLL = 8

                def blk_body(t, cur):
                    for b in range(2):
                        ch = 2 * t + b

                        def process(cur):
                            wait(b)

                            def vec_body(j, cur):
                                cur = lax.cond(
                                    cur > K_STG - UNROLL * LANES,
                                    flush_and_reset, lambda x: x, cur)
                                ivs, vvs, ms, mis = [], [], [], []
                                incls, pcs = [], []
                                for u in range(UNROLL):
                                    d = pl.ds((UNROLL * j + u) * LANES,
                                              LANES)
                                    ivs.append(rd_idx[b, d])
                                    vvs.append(rd_val[b, d])
                                locs = [iv - winoff_v for iv in ivs]
                                for u in range(UNROLL):
                                    m = (plsc.bitcast(locs[u], jnp.uint32)
                                         < wsz_u)
                                    ms.append(m)
                                    mis.append(m.astype(jnp.int32))
                                for u in range(UNROLL):
                                    incls.append(plsc.cumsum(mis[u]))
                                    pcs.append(
                                        plsc.all_reduce_population_count(
                                            ms[u]))
                                base = jnp.broadcast_to(
                                    cur, (LANES,)).astype(jnp.int32)
                                for u in range(UNROLL):
                                    offs = (base + incls[u]) - mis[u]
                                    plsc.store_scatter(
                                        stg_idx, [offs], locs[u],
                                        mask=ms[u])
                                    plsc.store_scatter(
                                        stg_val, [offs], vvs[u],
                                        mask=ms[u])
                                    base = base + pcs[u]
                                return base[0]

                            cur = lax.fori_loop(
                                0, KBLK // (UNROLL * LANES), vec_body, cur)

                            @pl.when(ch + 2 < tot)
                            def _():
                                issue(ch + 2, b)
                            return cur

                        cur = lax.cond(ch < tot, process,
                                       lambda x: x, cur)
                    return cur

                cur = lax.fori_loop(0, (tot + 1) // 2, blk_body, 0)
                flush_and_reset(cur)
                plsc.subcore_barrier()

                def oc(j, co):
                    off = s * SLICE + j * OCH
                    pltpu.sync_copy(acc.at[pl.ds(off, OCH)],
                                    out_hbm.at[pl.ds(wbase + off, OCH)])
                    return co
                lax.fori_loop(0, NO, oc, 0)

            return carry

        lax.fori_loop(0, (NWIN + NSC - 1) // NSC, win_body, 0)

    return accumulate


@jax.jit
def kernel(updates, mask):
    B, H, W, C = updates.shape
    out_h = H * _POOL[0]
    out_w = W * _POOL[1]
    M = B * out_h * out_w * C
    N = B * H * W * C
    p1 = _build_phase1(N, M)
    p2 = _build_phase2(N, M)
    sidx, sval, cnts = p1(mask.reshape(-1).astype(jnp.int32),
                          updates.reshape(-1))
    out = p2(sidx, sval, cnts)
    return out.reshape(-1, out_h, out_w, C)


# speedup vs baseline: 18.6653x; 1.0068x over previous
import functools

import jax
import jax.numpy as jnp
from jax import lax
from jax.experimental import pallas as pl
from jax.experimental.pallas import tpu as pltpu
from jax.experimental.pallas import tpu_sc as plsc

_POOL = (2, 2)

NSC = 2
NTILE = 16
NTG = 32
LANES = 16

WSZ = 1_835_008
NGRP = 7
WPG = 3
GRP = WSZ * WPG
KBLK = 2048
K_STG = 2048
C1 = 3072
GBUF = 5376
NBLK_CAP = 148
REG = NBLK_CAP * KBLK


def _mulshift_div(x, mul, shift):
    mv = jnp.full((LANES,), mul, jnp.int32)
    return lax.shift_right_logical(
        x * mv, jnp.full((LANES,), shift, jnp.int32))


@functools.lru_cache(maxsize=None)
def _build_phase1(N: int, M: int):
    NT32 = N // NTG
    assert NT32 * NTG == N
    NCH = NT32 // C1
    assert NCH * C1 == NT32 and NCH % 2 == 0
    SSZ = NGRP * NTG * REG

    mesh = plsc.VectorSubcoreMesh(core_axis_name="c", subcore_axis_name="s")

    @functools.partial(
        pl.kernel,
        out_type=(jax.ShapeDtypeStruct((SSZ,), jnp.int32),
                  jax.ShapeDtypeStruct((SSZ,), jnp.float32),
                  jax.ShapeDtypeStruct((NTG * LANES,), jnp.int32)),
        mesh=mesh,
        compiler_params=pltpu.CompilerParams(needs_layout_passes=False),
        scratch_types=[
            pltpu.VMEM((2, C1), jnp.int32),
            pltpu.VMEM((2, C1), jnp.float32),
            pltpu.VMEM((NGRP * GBUF,), jnp.int32),
            pltpu.VMEM((NGRP * GBUF,), jnp.float32),
            pltpu.VMEM((LANES,), jnp.int32),
            pltpu.SMEM((8,), jnp.int32),
            pltpu.SemaphoreType.DMA,
            pltpu.SemaphoreType.DMA,
            pltpu.SemaphoreType.DMA,
            pltpu.SemaphoreType.DMA,
        ],
    )
    def partition(mask_hbm, upd_hbm, sidx_hbm, sval_hbm, cnt_hbm,
                  in_idx, in_val, gb_idx, gb_val, cnt_v, blk_s,
                  s0i, s0v, s1i, s1v):
        c = lax.axis_index("c")
        s = lax.axis_index("s")
        tg = c * NTILE + s
        tbase = tg * NT32

        iota16 = lax.broadcasted_iota(jnp.int32, (LANES,), 0)
        dum_idx = iota16 + jnp.full((LANES,), GRP, jnp.int32)
        grp_v = jnp.full((LANES,), GRP, jnp.int32)

        for gg in range(NGRP):
            blk_s[gg] = 0

        sems = ((s0i, s0v), (s1i, s1v))

        def issue(chunk, slot):
            semi, semv = sems[slot]
            base = tbase + chunk * C1
            pltpu.async_copy(mask_hbm.at[pl.ds(base, C1)],
                             in_idx.at[slot], semi)
            pltpu.async_copy(upd_hbm.at[pl.ds(base, C1)],
                             in_val.at[slot], semv)

        def wait(slot):
            semi, semv = sems[slot]
            pltpu.make_async_copy(mask_hbm.at[pl.ds(0, C1)],
                                  in_idx.at[slot], semi).wait()
            pltpu.make_async_copy(upd_hbm.at[pl.ds(0, C1)],
                                  in_val.at[slot], semv).wait()

        issue(0, 0)
        issue(1, 1)

        def chunk_body(t, curs):
            for b in range(2):
                ch = 2 * t + b
                wait(b)

                def vec_body(j, curs):
                    U1 = 4
                    ivs, vvs, gvecs, glocals = [], [], [], []
                    for u in range(U1):
                        d = pl.ds((U1 * j + u) * LANES, LANES)
                        ivs.append(in_idx[b, d])
                        vvs.append(in_val[b, d])
                    for u in range(U1):
                        gv = _mulshift_div(
                            lax.shift_right_logical(
                                ivs[u], jnp.full((LANES,), 18, jnp.int32)),
                            3121, 16)
                        gvecs.append(gv)
                        glocals.append(ivs[u] - gv * grp_v)
                    newcurs = []
                    for gg in range(NGRP):
                        ggv = jnp.full((LANES,), gg, jnp.int32)
                        ms = [gvecs[u] == ggv for u in range(U1)]
                        mis = [m.astype(jnp.int32) for m in ms]
                        incls = [plsc.cumsum(mi) for mi in mis]
                        pcs = [plsc.all_reduce_population_count(m)
                               for m in ms]
                        base = curs[gg]
                        for u in range(U1):
                            offs = (base + incls[u]) - mis[u]
                            plsc.store_scatter(gb_idx, [offs], glocals[u],
                                               mask=ms[u])
                            plsc.store_scatter(gb_val, [offs], vvs[u],
                                               mask=ms[u])
                            base = base + pcs[u]
                        newcurs.append(base)
                    return tuple(newcurs)

                curs = lax.fori_loop(0, C1 // (4 * LANES), vec_body, curs)

                @pl.when(ch + 2 < NCH)
                def _():
                    issue(ch + 2, b)

                newcurs = []
                for gg in range(NGRP):
                    gb0 = gg * GBUF
                    cur = curs[gg][0] - gb0
                    rbase = (gg * NTG + tg) * REG

                    def flush_blk(f):
                        @pl.when(cur >= (f + 1) * KBLK)
                        def _():
                            nb = blk_s[gg]
                            pltpu.sync_copy(
                                gb_idx.at[pl.ds(gb0 + f * KBLK, KBLK)],
                                sidx_hbm.at[pl.ds(rbase + nb * KBLK, KBLK)])
                            pltpu.sync_copy(
                                gb_val.at[pl.ds(gb0 + f * KBLK, KBLK)],
                                sval_hbm.at[pl.ds(rbase + nb * KBLK, KBLK)])
                            blk_s[gg] = nb + 1

                    flush_blk(0)
                    flush_blk(1)
                    nf = cur // KBLK
                    rem = cur - nf * KBLK

                    @pl.when(nf > 0)
                    def _():
                        def mv(r, carry):
                            d = pl.ds(gb0 + nf * KBLK + r * LANES, LANES)
                            gb_idx[pl.ds(gb0 + r * LANES, LANES)] = \
                                gb_idx[d]
                            gb_val[pl.ds(gb0 + r * LANES, LANES)] = \
                                gb_val[d]
                            return carry
                        lax.fori_loop(0, (rem + LANES - 1) // LANES, mv, 0)

                    newcurs.append(jnp.broadcast_to(
                        gb0 + rem, (LANES,)).astype(jnp.int32))
                curs = tuple(newcurs)
            return curs

        zcur = tuple(
            jnp.full((LANES,), gg * GBUF, jnp.int32) for gg in range(NGRP))
        curs = lax.fori_loop(0, NCH // 2, chunk_body, zcur)

        for gg in range(NGRP):
            gb0 = gg * GBUF
            cur = curs[gg][0] - gb0
            rbase = (gg * NTG + tg) * REG
            gb_idx[pl.ds(gb0 + cur, LANES)] = dum_idx

            def pad(r, carry):
                gb_idx[pl.ds(gb0 + r * LANES, LANES)] = dum_idx
                return carry
            lax.fori_loop(cur // LANES + 1, KBLK // LANES, pad, 0)

            @pl.when(cur > 0)
            def _():
                nb = blk_s[gg]
                pltpu.sync_copy(gb_idx.at[pl.ds(gb0, KBLK)],
                                sidx_hbm.at[pl.ds(rbase + nb * KBLK, KBLK)])
                pltpu.sync_copy(gb_val.at[pl.ds(gb0, KBLK)],
                                sval_hbm.at[pl.ds(rbase + nb * KBLK, KBLK)])
                blk_s[gg] = nb + 1

        cnt_vec = jnp.zeros((LANES,), jnp.int32)
        for gg in range(NGRP):
            sel = iota16 == jnp.full((LANES,), gg, jnp.int32)
            cnt_vec = jnp.where(
                sel,
                jnp.broadcast_to(blk_s[gg], (LANES,)).astype(jnp.int32),
                cnt_vec)
        cnt_v[pl.ds(0, LANES)] = cnt_vec
        pltpu.sync_copy(cnt_v, cnt_hbm.at[pl.ds(tg * LANES, LANES)])

    return partition


@functools.lru_cache(maxsize=None)
def _build_phase2(N: int, M: int):
    NWIN = M // WSZ
    assert NWIN == NGRP * WPG
    SLICE = WSZ // NTILE
    ZCH = 2048
    NZ = SLICE // ZCH
    OCH = 16384
    NO = SLICE // OCH
    SSZ = NGRP * NTG * REG

    mesh = plsc.VectorSubcoreMesh(core_axis_name="c", subcore_axis_name="s")

    @functools.partial(
        pl.kernel,
        out_type=jax.ShapeDtypeStruct((M,), jnp.float32),
        mesh=mesh,
        compiler_params=pltpu.CompilerParams(needs_layout_passes=False),
        scratch_types=[
            pltpu.VMEM((2, KBLK), jnp.int32),
            pltpu.VMEM((2, KBLK), jnp.float32),
            pltpu.VMEM((K_STG,), jnp.int32),
            pltpu.VMEM((K_STG,), jnp.float32),
            pltpu.VMEM((2048,), jnp.float32),
            pltpu.VMEM((NTG * LANES,), jnp.int32),
            pltpu.VMEM_SHARED((WSZ + LANES,), jnp.float32),
            pltpu.SemaphoreType.DMA,
            pltpu.SemaphoreType.DMA,
            pltpu.SemaphoreType.DMA,
            pltpu.SemaphoreType.DMA,
        ],
    )
    def accumulate(sidx_hbm, sval_hbm, cnt_hbm, out_hbm,
                   rd_idx, rd_val, stg_idx, stg_val, zbuf, cnt_v, acc,
                   s0i, s0v, s1i, s1v):
        c = lax.axis_index("c")
        s = lax.axis_index("s")

        iota16 = lax.broadcasted_iota(jnp.int32, (LANES,), 0)
        wsz_i = jnp.full((LANES,), WSZ, jnp.int32)
        wsz_u = jnp.full((LANES,), WSZ, jnp.uint32)
        dum_idx = iota16 + wsz_i
        zvec = jnp.zeros((LANES,), jnp.float32)

        def zb(r, carry):
            zbuf[pl.ds(r * LANES, LANES)] = zvec
            return carry
        lax.fori_loop(0, 2048 // LANES, zb, 0)

        def refill(r, carry):
            stg_idx[pl.ds(r * LANES, LANES)] = dum_idx
            stg_val[pl.ds(r * LANES, LANES)] = zvec
            return carry
        lax.fori_loop(0, K_STG // LANES, refill, 0)

        pltpu.sync_copy(cnt_hbm, cnt_v)

        sems = ((s0i, s0v), (s1i, s1v))

        def flush_and_reset(_cur):
            pltpu.sync_copy(stg_val, acc.at[stg_idx], add=True)
            lax.fori_loop(0, K_STG // LANES, refill, 0)
            return 0

        def win_body(i, carry):
            w = c + NSC * i

            @pl.when(w < NWIN)
            def _run():
                wbase = w * WSZ
                g = (w * 10923) >> 15
                winoff = wbase - g * GRP
                winoff_v = jnp.broadcast_to(winoff, (LANES,)).astype(
                    jnp.int32)

                r0 = 2 * s
                gv = jnp.broadcast_to(g, (LANES,)).astype(jnp.int32)
                row0 = cnt_v[pl.ds(r0 * LANES, LANES)]
                row1 = cnt_v[pl.ds((r0 + 1) * LANES, LANES)]
                gsel = iota16 == gv
                zi = jnp.zeros((LANES,), jnp.int32)
                n0 = jnp.sum(jnp.where(gsel, row0, zi))
                n1 = jnp.sum(jnp.where(gsel, row1, zi))
                tot = n0 + n1
                a0 = (g * NTG + r0) * REG
                a1 = (g * NTG + r0 + 1) * REG

                def baddr(blk):
                    return jnp.where(blk < n0,
                                     a0 + blk * KBLK,
                                     a1 + (blk - n0) * KBLK)

                def issue(blk, slot):
                    semi, semv = sems[slot]
                    ba = baddr(blk)
                    pltpu.async_copy(sidx_hbm.at[pl.ds(ba, KBLK)],
                                     rd_idx.at[slot], semi)
                    pltpu.async_copy(sval_hbm.at[pl.ds(ba, KBLK)],
                                     rd_val.at[slot], semv)

                def wait(slot):
                    semi, semv = sems[slot]
                    pltpu.make_async_copy(sidx_hbm.at[pl.ds(0, KBLK)],
                                          rd_idx.at[slot], semi).wait()
                    pltpu.make_async_copy(sval_hbm.at[pl.ds(0, KBLK)],
                                          rd_val.at[slot], semv).wait()

                def zc(j, cz):
                    pltpu.sync_copy(
                        zbuf, acc.at[pl.ds(s * SLICE + j * ZCH, ZCH)])
                    return cz
                lax.fori_loop(0, NZ, zc, 0)

                @pl.when(tot > 0)
                def _():
                    issue(0, 0)

                @pl.when(tot > 1)
                def _():
                    issue(1, 1)

                plsc.subcore_barrier()

                UNROLL = 8

                def blk_body(t, cur):
                    for b in range(2):
                        ch = 2 * t + b

                        def process(cur):
                            wait(b)

                            def vec_body(j, cur):
                                cur = lax.cond(
                                    cur > K_STG - UNROLL * LANES,
                                    flush_and_reset, lambda x: x, cur)
                                ivs, vvs, ms, mis = [], [], [], []
                                incls, pcs = [], []
                                for u in range(UNROLL):
                                    d = pl.ds((UNROLL * j + u) * LANES,
                                              LANES)
                                    ivs.append(rd_idx[b, d])
                                    vvs.append(rd_val[b, d])
                                locs = [iv - winoff_v for iv in ivs]
                                for u in range(UNROLL):
                                    m = (plsc.bitcast(locs[u], jnp.uint32)
                                         < wsz_u)
                                    ms.append(m)
                                    mis.append(m.astype(jnp.int32))
                                for u in range(UNROLL):
                                    incls.append(plsc.cumsum(mis[u]))
                                    pcs.append(
                                        plsc.all_reduce_population_count(
                                            ms[u]))
                                base = jnp.broadcast_to(
                                    cur, (LANES,)).astype(jnp.int32)
                                for u in range(UNROLL):
                                    offs = (base + incls[u]) - mis[u]
                                    plsc.store_scatter(
                                        stg_idx, [offs], locs[u],
                                        mask=ms[u])
                                    plsc.store_scatter(
                                        stg_val, [offs], vvs[u],
                                        mask=ms[u])
                                    base = base + pcs[u]
                                return base[0]

                            cur = lax.fori_loop(
                                0, KBLK // (UNROLL * LANES), vec_body, cur)

                            @pl.when(ch + 2 < tot)
                            def _():
                                issue(ch + 2, b)
                            return cur

                        cur = lax.cond(ch < tot, process,
                                       lambda x: x, cur)
                    return cur

                cur = lax.fori_loop(0, (tot + 1) // 2, blk_body, 0)
                flush_and_reset(cur)
                plsc.subcore_barrier()

                def oc(j, co):
                    off = s * SLICE + j * OCH
                    pltpu.sync_copy(acc.at[pl.ds(off, OCH)],
                                    out_hbm.at[pl.ds(wbase + off, OCH)])
                    return co
                lax.fori_loop(0, NO, oc, 0)

            return carry

        lax.fori_loop(0, (NWIN + NSC - 1) // NSC, win_body, 0)

    return accumulate


@jax.jit
def kernel(updates, mask):
    B, H, W, C = updates.shape
    out_h = H * _POOL[0]
    out_w = W * _POOL[1]
    M = B * out_h * out_w * C
    N = B * H * W * C
    p1 = _build_phase1(N, M)
    p2 = _build_phase2(N, M)
    sidx, sval, cnts = p1(mask.reshape(-1).astype(jnp.int32),
                          updates.reshape(-1))
    out = p2(sidx, sval, cnts)
    return out.reshape(-1, out_h, out_w, C)


# speedup vs baseline: 18.8044x; 1.0075x over previous
import functools

import jax
import jax.numpy as jnp
from jax import lax
from jax.experimental import pallas as pl
from jax.experimental.pallas import tpu as pltpu
from jax.experimental.pallas import tpu_sc as plsc

_POOL = (2, 2)

NSC = 2
NTILE = 16
NTG = 32
LANES = 16

WSZ = 1_835_008
NGRP = 7
WPG = 3
GRP = WSZ * WPG
KBLK = 2048
K_STG = 2048
C1 = 3072
GBUF = 5376
NBLK_CAP = 148
REG = NBLK_CAP * KBLK


def _mulshift_div(x, mul, shift):
    mv = jnp.full((LANES,), mul, jnp.int32)
    return lax.shift_right_logical(
        x * mv, jnp.full((LANES,), shift, jnp.int32))


@functools.lru_cache(maxsize=None)
def _build_phase1(N: int, M: int):
    NT32 = N // NTG
    assert NT32 * NTG == N
    NCH = NT32 // C1
    assert NCH * C1 == NT32 and NCH % 2 == 0
    SSZ = NGRP * NTG * REG

    mesh = plsc.VectorSubcoreMesh(core_axis_name="c", subcore_axis_name="s")

    @functools.partial(
        pl.kernel,
        out_type=(jax.ShapeDtypeStruct((SSZ,), jnp.int32),
                  jax.ShapeDtypeStruct((SSZ,), jnp.float32),
                  jax.ShapeDtypeStruct((NTG * LANES,), jnp.int32)),
        mesh=mesh,
        compiler_params=pltpu.CompilerParams(needs_layout_passes=False),
        scratch_types=[
            pltpu.VMEM((2, C1), jnp.int32),
            pltpu.VMEM((2, C1), jnp.float32),
            pltpu.VMEM((NGRP * GBUF,), jnp.int32),
            pltpu.VMEM((NGRP * GBUF,), jnp.float32),
            pltpu.VMEM((LANES,), jnp.int32),
            pltpu.SMEM((8,), jnp.int32),
            pltpu.SemaphoreType.DMA,
            pltpu.SemaphoreType.DMA,
            pltpu.SemaphoreType.DMA,
            pltpu.SemaphoreType.DMA,
        ],
    )
    def partition(mask_hbm, upd_hbm, sidx_hbm, sval_hbm, cnt_hbm,
                  in_idx, in_val, gb_idx, gb_val, cnt_v, blk_s,
                  s0i, s0v, s1i, s1v):
        c = lax.axis_index("c")
        s = lax.axis_index("s")
        tg = c * NTILE + s
        tbase = tg * NT32

        iota16 = lax.broadcasted_iota(jnp.int32, (LANES,), 0)
        dum_idx = iota16 + jnp.full((LANES,), GRP, jnp.int32)
        grp_v = jnp.full((LANES,), GRP, jnp.int32)

        for gg in range(NGRP):
            blk_s[gg] = 0

        sems = ((s0i, s0v), (s1i, s1v))

        def issue(chunk, slot):
            semi, semv = sems[slot]
            base = tbase + chunk * C1
            pltpu.async_copy(mask_hbm.at[pl.ds(base, C1)],
                             in_idx.at[slot], semi)
            pltpu.async_copy(upd_hbm.at[pl.ds(base, C1)],
                             in_val.at[slot], semv)

        def wait(slot):
            semi, semv = sems[slot]
            pltpu.make_async_copy(mask_hbm.at[pl.ds(0, C1)],
                                  in_idx.at[slot], semi).wait()
            pltpu.make_async_copy(upd_hbm.at[pl.ds(0, C1)],
                                  in_val.at[slot], semv).wait()

        issue(0, 0)
        issue(1, 1)

        def chunk_body(t, curs):
            for b in range(2):
                ch = 2 * t + b
                wait(b)

                def vec_body(j, curs):
                    U1 = 4
                    ivs, vvs, gvecs, glocals = [], [], [], []
                    for u in range(U1):
                        d = pl.ds((U1 * j + u) * LANES, LANES)
                        ivs.append(in_idx[b, d])
                        vvs.append(in_val[b, d])
                    for u in range(U1):
                        gv = _mulshift_div(
                            lax.shift_right_logical(
                                ivs[u], jnp.full((LANES,), 18, jnp.int32)),
                            3121, 16)
                        gvecs.append(gv)
                        glocals.append(ivs[u] - gv * grp_v)
                    newcurs = []
                    for gg in range(NGRP):
                        ggv = jnp.full((LANES,), gg, jnp.int32)
                        ms = [gvecs[u] == ggv for u in range(U1)]
                        mis = [m.astype(jnp.int32) for m in ms]
                        incls = [plsc.cumsum(mi) for mi in mis]
                        pcs = [plsc.all_reduce_population_count(m)
                               for m in ms]
                        base = curs[gg]
                        for u in range(U1):
                            offs = (base + incls[u]) - mis[u]
                            plsc.store_scatter(gb_idx, [offs], glocals[u],
                                               mask=ms[u])
                            plsc.store_scatter(gb_val, [offs], vvs[u],
                                               mask=ms[u])
                            base = base + pcs[u]
                        newcurs.append(base)
                    return tuple(newcurs)

                curs = lax.fori_loop(0, C1 // (4 * LANES), vec_body, curs)

                @pl.when(ch + 2 < NCH)
                def _():
                    issue(ch + 2, b)

                newcurs = []
                for gg in range(NGRP):
                    gb0 = gg * GBUF
                    cur = curs[gg][0] - gb0
                    rbase = (gg * NTG + tg) * REG

                    def flush_blk(f):
                        @pl.when(cur >= (f + 1) * KBLK)
                        def _():
                            nb = blk_s[gg]
                            pltpu.sync_copy(
                                gb_idx.at[pl.ds(gb0 + f * KBLK, KBLK)],
                                sidx_hbm.at[pl.ds(rbase + nb * KBLK, KBLK)])
                            pltpu.sync_copy(
                                gb_val.at[pl.ds(gb0 + f * KBLK, KBLK)],
                                sval_hbm.at[pl.ds(rbase + nb * KBLK, KBLK)])
                            blk_s[gg] = nb + 1

                    flush_blk(0)
                    flush_blk(1)
                    nf = cur // KBLK
                    rem = cur - nf * KBLK

                    @pl.when(nf > 0)
                    def _():
                        def mv(r, carry):
                            d = pl.ds(gb0 + nf * KBLK + r * LANES, LANES)
                            gb_idx[pl.ds(gb0 + r * LANES, LANES)] = \
                                gb_idx[d]
                            gb_val[pl.ds(gb0 + r * LANES, LANES)] = \
                                gb_val[d]
                            return carry
                        lax.fori_loop(0, (rem + LANES - 1) // LANES, mv, 0)

                    newcurs.append(jnp.broadcast_to(
                        gb0 + rem, (LANES,)).astype(jnp.int32))
                curs = tuple(newcurs)
            return curs

        zcur = tuple(
            jnp.full((LANES,), gg * GBUF, jnp.int32) for gg in range(NGRP))
        curs = lax.fori_loop(0, NCH // 2, chunk_body, zcur)

        for gg in range(NGRP):
            gb0 = gg * GBUF
            cur = curs[gg][0] - gb0
            rbase = (gg * NTG + tg) * REG
            gb_idx[pl.ds(gb0 + cur, LANES)] = dum_idx

            def pad(r, carry):
                gb_idx[pl.ds(gb0 + r * LANES, LANES)] = dum_idx
                return carry
            lax.fori_loop(cur // LANES + 1, KBLK // LANES, pad, 0)

            @pl.when(cur > 0)
            def _():
                nb = blk_s[gg]
                pltpu.sync_copy(gb_idx.at[pl.ds(gb0, KBLK)],
                                sidx_hbm.at[pl.ds(rbase + nb * KBLK, KBLK)])
                pltpu.sync_copy(gb_val.at[pl.ds(gb0, KBLK)],
                                sval_hbm.at[pl.ds(rbase + nb * KBLK, KBLK)])
                blk_s[gg] = nb + 1

        cnt_vec = jnp.zeros((LANES,), jnp.int32)
        for gg in range(NGRP):
            sel = iota16 == jnp.full((LANES,), gg, jnp.int32)
            cnt_vec = jnp.where(
                sel,
                jnp.broadcast_to(blk_s[gg], (LANES,)).astype(jnp.int32),
                cnt_vec)
        cnt_v[pl.ds(0, LANES)] = cnt_vec
        pltpu.sync_copy(cnt_v, cnt_hbm.at[pl.ds(tg * LANES, LANES)])

    return partition


@functools.lru_cache(maxsize=None)
def _build_phase2(N: int, M: int):
    NWIN = M // WSZ
    assert NWIN == NGRP * WPG
    SLICE = WSZ // NTILE
    ZCH = 2048
    NZ = SLICE // ZCH
    OCH = 16384
    NO = SLICE // OCH
    SSZ = NGRP * NTG * REG

    mesh = plsc.VectorSubcoreMesh(core_axis_name="c", subcore_axis_name="s")

    @functools.partial(
        pl.kernel,
        out_type=jax.ShapeDtypeStruct((M,), jnp.float32),
        mesh=mesh,
        compiler_params=pltpu.CompilerParams(needs_layout_passes=False),
        scratch_types=[
            pltpu.VMEM((2, KBLK), jnp.int32),
            pltpu.VMEM((2, KBLK), jnp.float32),
            pltpu.VMEM((K_STG,), jnp.int32),
            pltpu.VMEM((K_STG,), jnp.float32),
            pltpu.VMEM((2048,), jnp.float32),
            pltpu.VMEM((NTG * LANES,), jnp.int32),
            pltpu.VMEM_SHARED((WSZ + LANES,), jnp.float32),
            pltpu.SemaphoreType.DMA,
            pltpu.SemaphoreType.DMA,
            pltpu.SemaphoreType.DMA,
            pltpu.SemaphoreType.DMA,
        ],
    )
    def accumulate(sidx_hbm, sval_hbm, cnt_hbm, out_hbm,
                   rd_idx, rd_val, stg_idx, stg_val, zbuf, cnt_v, acc,
                   s0i, s0v, s1i, s1v):
        c = lax.axis_index("c")
        s = lax.axis_index("s")

        iota16 = lax.broadcasted_iota(jnp.int32, (LANES,), 0)
        wsz_i = jnp.full((LANES,), WSZ, jnp.int32)
        wsz_u = jnp.full((LANES,), WSZ, jnp.uint32)
        dum_idx = iota16 + wsz_i
        zvec = jnp.zeros((LANES,), jnp.float32)

        def zb(r, carry):
            zbuf[pl.ds(r * LANES, LANES)] = zvec
            return carry
        lax.fori_loop(0, 2048 // LANES, zb, 0)

        def refill(r, carry):
            stg_idx[pl.ds(r * LANES, LANES)] = dum_idx
            stg_val[pl.ds(r * LANES, LANES)] = zvec
            return carry
        lax.fori_loop(0, K_STG // LANES, refill, 0)

        pltpu.sync_copy(cnt_hbm, cnt_v)

        sems = ((s0i, s0v), (s1i, s1v))

        def flush_and_reset(_cur):
            pltpu.sync_copy(stg_val, acc.at[stg_idx], add=True)
            lax.fori_loop(0, K_STG // LANES, refill, 0)
            return 0

        def win_body(i, carry):
            w = c + NSC * i

            @pl.when(w < NWIN)
            def _run():
                wbase = w * WSZ
                g = (w * 10923) >> 15
                winoff = wbase - g * GRP
                winoff_v = jnp.broadcast_to(winoff, (LANES,)).astype(
                    jnp.int32)

                r0 = 2 * s
                gv = jnp.broadcast_to(g, (LANES,)).astype(jnp.int32)
                row0 = cnt_v[pl.ds(r0 * LANES, LANES)]
                row1 = cnt_v[pl.ds((r0 + 1) * LANES, LANES)]
                gsel = iota16 == gv
                zi = jnp.zeros((LANES,), jnp.int32)
                n0 = jnp.sum(jnp.where(gsel, row0, zi))
                n1 = jnp.sum(jnp.where(gsel, row1, zi))
                tot = n0 + n1
                a0 = (g * NTG + r0) * REG
                a1 = (g * NTG + r0 + 1) * REG

                def baddr(blk):
                    return jnp.where(blk < n0,
                                     a0 + blk * KBLK,
                                     a1 + (blk - n0) * KBLK)

                def issue(blk, slot):
                    semi, semv = sems[slot]
                    ba = baddr(blk)
                    pltpu.async_copy(sidx_hbm.at[pl.ds(ba, KBLK)],
                                     rd_idx.at[slot], semi)
                    pltpu.async_copy(sval_hbm.at[pl.ds(ba, KBLK)],
                                     rd_val.at[slot], semv)

                def wait(slot):
                    semi, semv = sems[slot]
                    pltpu.make_async_copy(sidx_hbm.at[pl.ds(0, KBLK)],
                                          rd_idx.at[slot], semi).wait()
                    pltpu.make_async_copy(sval_hbm.at[pl.ds(0, KBLK)],
                                          rd_val.at[slot], semv).wait()

                def zc(j, cz):
                    pltpu.sync_copy(
                        zbuf, acc.at[pl.ds(s * SLICE + j * ZCH, ZCH)])
                    return cz
                lax.fori_loop(0, NZ, zc, 0)

                @pl.when(tot > 0)
                def _():
                    issue(0, 0)

                @pl.when(tot > 1)
                def _():
                    issue(1, 1)

                plsc.subcore_barrier()

                UNROLL = 16

                def blk_body(t, cur):
                    for b in range(2):
                        ch = 2 * t + b

                        def process(cur):
                            wait(b)

                            def vec_body(j, cur):
                                cur = lax.cond(
                                    cur > K_STG - UNROLL * LANES,
                                    flush_and_reset, lambda x: x, cur)
                                ivs, vvs, ms, mis = [], [], [], []
                                incls, pcs = [], []
                                for u in range(UNROLL):
                                    d = pl.ds((UNROLL * j + u) * LANES,
                                              LANES)
                                    ivs.append(rd_idx[b, d])
                                    vvs.append(rd_val[b, d])
                                locs = [iv - winoff_v for iv in ivs]
                                for u in range(UNROLL):
                                    m = (plsc.bitcast(locs[u], jnp.uint32)
                                         < wsz_u)
                                    ms.append(m)
                                    mis.append(m.astype(jnp.int32))
                                for u in range(UNROLL):
                                    incls.append(plsc.cumsum(mis[u]))
                                    pcs.append(
                                        plsc.all_reduce_population_count(
                                            ms[u]))
                                base = jnp.broadcast_to(
                                    cur, (LANES,)).astype(jnp.int32)
                                for u in range(UNROLL):
                                    offs = (base + incls[u]) - mis[u]
                                    plsc.store_scatter(
                                        stg_idx, [offs], locs[u],
                                        mask=ms[u])
                                    plsc.store_scatter(
                                        stg_val, [offs], vvs[u],
                                        mask=ms[u])
                                    base = base + pcs[u]
                                return base[0]

                            cur = lax.fori_loop(
                                0, KBLK // (UNROLL * LANES), vec_body, cur)

                            @pl.when(ch + 2 < tot)
                            def _():
                                issue(ch + 2, b)
                            return cur

                        cur = lax.cond(ch < tot, process,
                                       lambda x: x, cur)
                    return cur

                cur = lax.fori_loop(0, (tot + 1) // 2, blk_body, 0)
                flush_and_reset(cur)
                plsc.subcore_barrier()

                def oc(j, co):
                    off = s * SLICE + j * OCH
                    pltpu.sync_copy(acc.at[pl.ds(off, OCH)],
                                    out_hbm.at[pl.ds(wbase + off, OCH)])
                    return co
                lax.fori_loop(0, NO, oc, 0)

            return carry

        lax.fori_loop(0, (NWIN + NSC - 1) // NSC, win_body, 0)

    return accumulate


@jax.jit
def kernel(updates, mask):
    B, H, W, C = updates.shape
    out_h = H * _POOL[0]
    out_w = W * _POOL[1]
    M = B * out_h * out_w * C
    N = B * H * W * C
    p1 = _build_phase1(N, M)
    p2 = _build_phase2(N, M)
    sidx, sval, cnts = p1(mask.reshape(-1).astype(jnp.int32),
                          updates.reshape(-1))
    out = p2(sidx, sval, cnts)
    return out.reshape(-1, out_h, out_w, C)
